# Initial kernel scaffold; baseline (speedup 1.0000x reference)
import functools

import jax
import jax.numpy as jnp
from jax import lax
from jax.experimental import pallas as pl
from jax.experimental.pallas import tpu as pltpu
from jax.experimental.pallas import tpu_sc as plsc

F32 = jnp.float32

D = 80
WIN = 640
CHUNK = 128
NC, NS = 2, 16
NWORK = NC * NS

BN = 2000
BE = 1024


def _silu(v):
    return v * jax.nn.sigmoid(v)


def _emb_body(x_ref, w_ref, b_ref, o_ref):
    o_ref[...] = jnp.dot(x_ref[...], w_ref[...],
                         preferred_element_type=F32) + b_ref[...]


def _node_pre_body(h_ref, cp_ref, w1a_ref, w1b_ref, b1_ref, tr_ref, tc_ref):
    h = h_ref[...]
    cp = cp_ref[...]
    u = jnp.dot(h, w1a_ref[...], preferred_element_type=F32) + b1_ref[...]
    v = jnp.dot(h, w1b_ref[...], preferred_element_type=F32)
    tr_ref[...] = jnp.concatenate([u, cp], axis=1)
    tc_ref[...] = jnp.concatenate([v, cp], axis=1)


def _edge_body(n_edges, gr_ref, gc_ref, ea_ref, wr_ref, w1d_ref, b2_ref,
               w2_ref, cw1_ref, cb1_ref, cw2_ref, o_ref):
    gr = gr_ref[...]
    gc = gc_ref[...]
    cd = gr[:, 64:80] - gc[:, 64:80]
    radial = jnp.sum(cd * cd, axis=1, keepdims=True)
    pre = (gr[:, :64] + gc[:, :64] + radial * wr_ref[...] +
           jnp.dot(ea_ref[...], w1d_ref[...], preferred_element_type=F32))
    m = _silu(jnp.dot(_silu(pre), w2_ref[...],
                      preferred_element_type=F32) + b2_ref[...])
    p = _silu(jnp.dot(m, cw1_ref[...], preferred_element_type=F32) + cb1_ref[...])
    cmat = jnp.sum(p * cw2_ref[...], axis=1, keepdims=True)
    trans = jnp.clip(cmat * cd, -100.0, 100.0)
    lane16 = lax.broadcasted_iota(jnp.int32, (1, 16), 1)
    tpack = jnp.where(lane16 < 3, trans, 0.0)
    tpack = jnp.where(lane16 == 3, 1.0, tpack)
    base = pl.program_id(0) * gr.shape[0]
    valid = (lax.broadcasted_iota(jnp.int32, (gr.shape[0], 1), 0) + base
             < n_edges).astype(F32)
    o_ref[...] = jnp.concatenate([m, tpack], axis=1) * valid


def _node_post_body(s_ref, h_ref, cp_ref, vp_ref, vw1_ref, vb1_ref, vw2_ref,
                    vb2_ref, nw1h_ref, nw1a_ref, nb1_ref, nw2_ref, nb2_ref,
                    ho_ref, co_ref):
    s = s_ref[...]
    h = h_ref[...]
    cnt = jnp.clip(s[:, 67:68], 1.0, None)
    agg = s[:, :64] / cnt
    lane16 = lax.broadcasted_iota(jnp.int32, (1, 16), 1)
    dcoord = jnp.where(lane16 < 3, s[:, 64:80], 0.0) / cnt
    sv = _silu(jnp.dot(h, vw1_ref[...], preferred_element_type=F32) + vb1_ref[...])
    vmat = jnp.sum(sv * vw2_ref[...], axis=1, keepdims=True) + vb2_ref[...]
    co_ref[...] = cp_ref[...] + dcoord + vmat * vp_ref[...]
    z = _silu(jnp.dot(h, nw1h_ref[...], preferred_element_type=F32) +
              jnp.dot(agg, nw1a_ref[...], preferred_element_type=F32) +
              nb1_ref[...])
    ho_ref[...] = h + jnp.dot(z, nw2_ref[...],
                              preferred_element_type=F32) + nb2_ref[...]


def _head_body(h_ref, cp_ref, vp_ref,
               ah1_ref, ac1_ref, av1_ref, ab1_ref, aw2_ref, ab2_ref,
               aw3_ref, ab3_ref,
               bh1_ref, bc1_ref, bv1_ref, bb1_ref, bw2_ref, bb2_ref,
               bw3_ref, bb3_ref, o_ref):
    h = h_ref[...]
    cp = cp_ref[...]
    vp = vp_ref[...]

    def head(h1, c1, v1, b1, w2, b2, w3, b3):
        z = _silu(jnp.dot(h, h1, preferred_element_type=F32) +
                  jnp.dot(cp, c1, preferred_element_type=F32) +
                  jnp.dot(vp, v1, preferred_element_type=F32) + b1)
        z = _silu(jnp.dot(z, w2, preferred_element_type=F32) + b2)
        return jnp.dot(z, w3, preferred_element_type=F32) + b3

    oa = head(ah1_ref[...], ac1_ref[...], av1_ref[...], ab1_ref[...],
              aw2_ref[...], ab2_ref[...], aw3_ref[...], ab3_ref[...])
    ob = head(bh1_ref[...], bc1_ref[...], bv1_ref[...], bb1_ref[...],
              bw2_ref[...], bb2_ref[...], bw3_ref[...], bb3_ref[...])
    o_ref[...] = jnp.concatenate([oa, ob], axis=1)


def _tc_call(body, grid, in_specs, out_specs, out_shape):
    return pl.pallas_call(body, grid=grid, in_specs=in_specs,
                          out_specs=out_specs, out_shape=out_shape)


def _row_spec(b, d):
    return pl.BlockSpec((b, d), lambda i: (i, 0))


def _full_spec(s0, s1):
    return pl.BlockSpec((s0, s1), lambda i: (0, 0))


def _sc_gather_body(epad, tr_hbm, tc_hbm, row_hbm, col_hbm, gr_hbm, gc_hbm,
                    idx_v, rows_v, sem):
    core = lax.axis_index("c")
    sub = lax.axis_index("s")
    wid = sub * NC + core
    nwin = epad // WIN
    k = WIN // CHUNK

    @pl.loop(0, nwin // NWORK)
    def _(i):
        win = wid + i * NWORK

        def one_side(src_idx, table, out):
            pltpu.sync_copy(src_idx.at[pl.ds(win * k, k)], idx_v)
            cps = [pltpu.async_copy(table.at[idx_v.at[j]],
                                    rows_v.at[pl.ds(j * CHUNK, CHUNK)], sem)
                   for j in range(k)]
            for cp in cps:
                cp.wait()
            pltpu.sync_copy(rows_v, out.at[pl.ds(win * WIN, WIN)])

        one_side(row_hbm, tr_hbm, gr_hbm)
        one_side(col_hbm, tc_hbm, gc_hbm)


def _sc_scatter_body(n_half, acc_rows, gout_hbm, row_hbm, s_hbm,
                     idx_v, upd_v, zero_v, acc, sem):
    core = lax.axis_index("c")
    sub = lax.axis_index("s")
    base = core * n_half
    nwin = gout_hbm.shape[0] // WIN
    k = WIN // CHUNK
    nchunk = acc_rows // CHUNK
    vz = jnp.zeros((16,), F32)
    iota = lax.iota(jnp.int32, 16)

    @pl.loop(0, CHUNK)
    def _(r):
        @pl.loop(0, D // 16)
        def _(c):
            zero_v[r, pl.ds(c * 16, 16)] = vz

    @pl.loop(0, pl.cdiv(nchunk, NS))
    def _(i):
        c = sub + i * NS

        @pl.when(c < nchunk)
        def _():
            pltpu.sync_copy(zero_v, acc.at[pl.ds(c * CHUNK, CHUNK)])

    plsc.subcore_barrier()

    @pl.loop(0, nwin // NS)
    def _(i):
        win = sub + i * NS
        pltpu.sync_copy(row_hbm.at[pl.ds(win * k, k)], idx_v)

        @pl.loop(0, k)
        def _(j):
            @pl.loop(0, CHUNK // 16)
            def _(t):
                v = idx_v[j, pl.ds(t * 16, 16)]
                local = v - base
                oob = (local < 0) | (local >= n_half)
                dump = n_half + ((j * (CHUNK // 16) + t) % 5) * 16 + iota
                idx_v[j, pl.ds(t * 16, 16)] = jnp.where(oob, dump, local)

        pltpu.sync_copy(gout_hbm.at[pl.ds(win * WIN, WIN)], upd_v)
        for j in range(k):
            pltpu.sync_copy(upd_v.at[pl.ds(j * CHUNK, CHUNK)],
                            acc.at[idx_v.at[j]], add=True)

    plsc.subcore_barrier()

    nfull = n_half // CHUNK
    rem = n_half - nfull * CHUNK

    @pl.loop(0, pl.cdiv(nfull, NS))
    def _(i):
        c = sub + i * NS

        @pl.when(c < nfull)
        def _():
            pltpu.sync_copy(acc.at[pl.ds(c * CHUNK, CHUNK)],
                            s_hbm.at[pl.ds(base + c * CHUNK, CHUNK)])

    if rem:
        @pl.when(sub == 0)
        def _():
            pltpu.sync_copy(acc.at[pl.ds(nfull * CHUNK, rem)],
                            s_hbm.at[pl.ds(base + nfull * CHUNK, rem)])


def kernel(x, pos, vel, edge_index, edge_attr, emb_W, emb_b,
           edge_W1, edge_b1, edge_W2, edge_b2,
           node_W1, node_b1, node_W2, node_b2,
           coord_W1, coord_b1, coord_W2,
           vel_W1, vel_b1, vel_W2, vel_b2,
           head_W1, head_b1, head_W2, head_b2, head_W3, head_b3):
    n, din = x.shape
    hdim = emb_W.shape[1]
    e = edge_index.shape[1]
    nlayers = edge_W1.shape[0]
    nheads = head_W1.shape[0]

    stride = WIN * NWORK
    epad = pl.cdiv(e, stride) * stride
    n_half = pl.cdiv(n, NC)
    acc_rows = pl.cdiv(n_half + CHUNK, CHUNK) * CHUNK

    rowp = jnp.pad(edge_index[0], (0, epad - e)).reshape(-1, CHUNK)
    colp = jnp.pad(edge_index[1], (0, epad - e)).reshape(-1, CHUNK)
    eap = jnp.pad(edge_attr, ((0, epad - e), (0, 0)))
    coordp = jnp.pad(pos, ((0, 0), (0, 16 - pos.shape[1])))
    velp = jnp.pad(vel, ((0, 0), (0, 16 - vel.shape[1])))

    gn = pl.cdiv(n, BN)
    ge = epad // BE

    h = _tc_call(_emb_body, (gn,),
                 [_row_spec(BN, din), _full_spec(din, hdim),
                  _full_spec(1, hdim)],
                 _row_spec(BN, hdim),
                 jax.ShapeDtypeStruct((n, hdim), F32))(
                     x, emb_W, emb_b.reshape(1, hdim))

    mesh = plsc.VectorSubcoreMesh(core_axis_name="c", subcore_axis_name="s")
    sc_gather = pl.kernel(
        functools.partial(_sc_gather_body, epad),
        out_type=[jax.ShapeDtypeStruct((epad, D), F32),
                  jax.ShapeDtypeStruct((epad, D), F32)],
        mesh=mesh,
        scratch_types=[pltpu.VMEM((WIN // CHUNK, CHUNK), jnp.int32),
                       pltpu.VMEM((WIN, D), F32),
                       pltpu.SemaphoreType.DMA])
    sc_scatter = pl.kernel(
        functools.partial(_sc_scatter_body, n_half, acc_rows),
        out_type=jax.ShapeDtypeStruct((n, D), F32),
        mesh=mesh,
        scratch_types=[pltpu.VMEM((WIN // CHUNK, CHUNK), jnp.int32),
                       pltpu.VMEM((WIN, D), F32),
                       pltpu.VMEM((CHUNK, D), F32),
                       pltpu.VMEM_SHARED((acc_rows, D), F32),
                       pltpu.SemaphoreType.DMA])

    for l in range(nlayers):
        w1 = edge_W1[l]
        w1a, w1b = w1[:hdim], w1[hdim:2 * hdim]
        wr = w1[2 * hdim:2 * hdim + 1]
        w1d = w1[2 * hdim + 1:]

        tr, tc = _tc_call(
            _node_pre_body, (gn,),
            [_row_spec(BN, hdim), _row_spec(BN, 16), _full_spec(hdim, hdim),
             _full_spec(hdim, hdim), _full_spec(1, hdim)],
            [_row_spec(BN, D), _row_spec(BN, D)],
            [jax.ShapeDtypeStruct((n, D), F32),
             jax.ShapeDtypeStruct((n, D), F32)])(
                 h, coordp, w1a, w1b, edge_b1[l].reshape(1, hdim))

        gr, gc = sc_gather(tr, tc, rowp, colp)

        gout = _tc_call(
            functools.partial(_edge_body, e), (ge,),
            [_row_spec(BE, D), _row_spec(BE, D), _row_spec(BE, eap.shape[1]),
             _full_spec(1, hdim), _full_spec(eap.shape[1], hdim),
             _full_spec(1, hdim), _full_spec(hdim, hdim),
             _full_spec(hdim, hdim), _full_spec(1, hdim), _full_spec(1, hdim)],
            _row_spec(BE, D),
            jax.ShapeDtypeStruct((epad, D), F32))(
                gr, gc, eap, wr, w1d, edge_b2[l].reshape(1, hdim),
                edge_W2[l], coord_W1[l], coord_b1[l].reshape(1, hdim),
                coord_W2[l].reshape(1, hdim))

        s = sc_scatter(gout, rowp)

        h, coordp = _tc_call(
            _node_post_body, (gn,),
            [_row_spec(BN, D), _row_spec(BN, hdim), _row_spec(BN, 16),
             _row_spec(BN, 16), _full_spec(hdim, hdim), _full_spec(1, hdim),
             _full_spec(1, hdim), _full_spec(1, 1), _full_spec(hdim, hdim),
             _full_spec(hdim, hdim), _full_spec(1, hdim),
             _full_spec(hdim, hdim), _full_spec(1, hdim)],
            [_row_spec(BN, hdim), _row_spec(BN, 16)],
            [jax.ShapeDtypeStruct((n, hdim), F32),
             jax.ShapeDtypeStruct((n, 16), F32)])(
                s, h, coordp, velp,
                vel_W1[l], vel_b1[l].reshape(1, hdim),
                vel_W2[l].reshape(1, hdim), vel_b2[l].reshape(1, 1),
                node_W1[l][:hdim], node_W1[l][hdim:],
                node_b1[l].reshape(1, hdim), node_W2[l],
                node_b2[l].reshape(1, hdim))

    def hw(t):
        w1 = head_W1[t]
        h1 = w1[:hdim]
        c1 = jnp.pad(w1[hdim:hdim + 3], ((0, 13), (0, 0)))
        v1 = jnp.pad(w1[hdim + 3:hdim + 6], ((0, 13), (0, 0)))
        w3 = jnp.pad(head_W3[t], ((0, 0), (0, 5)))
        b3 = jnp.pad(head_b3[t], (0, 5)).reshape(1, 8)
        return (h1, c1, v1, head_b1[t].reshape(1, hdim), head_W2[t],
                head_b2[t].reshape(1, hdim), w3, b3)

    wspecs = [_full_spec(hdim, hdim), _full_spec(16, hdim),
              _full_spec(16, hdim), _full_spec(1, hdim),
              _full_spec(hdim, hdim), _full_spec(1, hdim),
              _full_spec(hdim, 8), _full_spec(1, 8)]
    out = _tc_call(
        _head_body, (gn,),
        [_row_spec(BN, hdim), _row_spec(BN, 16), _row_spec(BN, 16)]
        + wspecs + wspecs,
        _row_spec(BN, 16),
        jax.ShapeDtypeStruct((n, 16), F32))(
            h, coordp, velp, *hw(0), *hw(1))

    return out.reshape(n, nheads, 8)[:, :, :3].transpose(1, 0, 2)


# speedup vs baseline: 2.3771x; 2.3771x over previous
import functools

import jax
import jax.numpy as jnp
from jax import lax
from jax.experimental import pallas as pl
from jax.experimental.pallas import tpu as pltpu
from jax.experimental.pallas import tpu_sc as plsc

F32 = jnp.float32

TD = 128
D = 80
WIN = 1024
SWIN = 512
CHUNK = 128
NC, NS = 2, 16
NWORK = NC * NS

BN = 2000
BE = 1024


def _silu(v):
    return v * jax.nn.sigmoid(v)


def _emb_body(x_ref, w_ref, b_ref, o_ref):
    o_ref[...] = jnp.dot(x_ref[...], w_ref[...],
                         preferred_element_type=F32) + b_ref[...]


def _node_pre_body(h_ref, cp_ref, w1a_ref, w1b_ref, b1_ref, tr_ref, tc_ref):
    h = h_ref[...]
    cp = cp_ref[...]
    z = jnp.zeros((h.shape[0], TD - 80), F32)
    u = jnp.dot(h, w1a_ref[...], preferred_element_type=F32) + b1_ref[...]
    v = jnp.dot(h, w1b_ref[...], preferred_element_type=F32)
    tr_ref[...] = jnp.concatenate([u, cp, z], axis=1)
    tc_ref[...] = jnp.concatenate([v, cp, z], axis=1)


def _edge_body(n_edges, gr_ref, gc_ref, ea_ref, wr_ref, w1d_ref, b2_ref,
               w2_ref, cw1_ref, cb1_ref, cw2_ref, o_ref):
    gr = gr_ref[...]
    gc = gc_ref[...]
    cd = gr[:, 64:80] - gc[:, 64:80]
    radial = jnp.sum(cd * cd, axis=1, keepdims=True)
    pre = (gr[:, :64] + gc[:, :64] + radial * wr_ref[...] +
           jnp.dot(ea_ref[...], w1d_ref[...], preferred_element_type=F32))
    m = _silu(jnp.dot(_silu(pre), w2_ref[...],
                      preferred_element_type=F32) + b2_ref[...])
    p = _silu(jnp.dot(m, cw1_ref[...], preferred_element_type=F32) + cb1_ref[...])
    cmat = jnp.sum(p * cw2_ref[...], axis=1, keepdims=True)
    trans = jnp.clip(cmat * cd, -100.0, 100.0)
    lane16 = lax.broadcasted_iota(jnp.int32, (1, 16), 1)
    tpack = jnp.where(lane16 < 3, trans, 0.0)
    tpack = jnp.where(lane16 == 3, 1.0, tpack)
    base = pl.program_id(0) * gr.shape[0]
    valid = (lax.broadcasted_iota(jnp.int32, (gr.shape[0], 1), 0) + base
             < n_edges).astype(F32)
    o_ref[...] = jnp.concatenate([m, tpack], axis=1) * valid


def _node_post_body(s_ref, h_ref, cp_ref, vp_ref, vw1_ref, vb1_ref, vw2_ref,
                    vb2_ref, nw1h_ref, nw1a_ref, nb1_ref, nw2_ref, nb2_ref,
                    ho_ref, co_ref):
    s = s_ref[...]
    h = h_ref[...]
    cnt = jnp.clip(s[:, 67:68], 1.0, None)
    agg = s[:, :64] / cnt
    lane16 = lax.broadcasted_iota(jnp.int32, (1, 16), 1)
    dcoord = jnp.where(lane16 < 3, s[:, 64:80], 0.0) / cnt
    sv = _silu(jnp.dot(h, vw1_ref[...], preferred_element_type=F32) + vb1_ref[...])
    vmat = jnp.sum(sv * vw2_ref[...], axis=1, keepdims=True) + vb2_ref[...]
    co_ref[...] = cp_ref[...] + dcoord + vmat * vp_ref[...]
    z = _silu(jnp.dot(h, nw1h_ref[...], preferred_element_type=F32) +
              jnp.dot(agg, nw1a_ref[...], preferred_element_type=F32) +
              nb1_ref[...])
    ho_ref[...] = h + jnp.dot(z, nw2_ref[...],
                              preferred_element_type=F32) + nb2_ref[...]


def _head_body(h_ref, cp_ref, vp_ref,
               ah1_ref, ac1_ref, av1_ref, ab1_ref, aw2_ref, ab2_ref,
               aw3_ref, ab3_ref,
               bh1_ref, bc1_ref, bv1_ref, bb1_ref, bw2_ref, bb2_ref,
               bw3_ref, bb3_ref, o_ref):
    h = h_ref[...]
    cp = cp_ref[...]
    vp = vp_ref[...]

    def head(h1, c1, v1, b1, w2, b2, w3, b3):
        z = _silu(jnp.dot(h, h1, preferred_element_type=F32) +
                  jnp.dot(cp, c1, preferred_element_type=F32) +
                  jnp.dot(vp, v1, preferred_element_type=F32) + b1)
        z = _silu(jnp.dot(z, w2, preferred_element_type=F32) + b2)
        return jnp.dot(z, w3, preferred_element_type=F32) + b3

    oa = head(ah1_ref[...], ac1_ref[...], av1_ref[...], ab1_ref[...],
              aw2_ref[...], ab2_ref[...], aw3_ref[...], ab3_ref[...])
    ob = head(bh1_ref[...], bc1_ref[...], bv1_ref[...], bb1_ref[...],
              bw2_ref[...], bb2_ref[...], bw3_ref[...], bb3_ref[...])
    o_ref[...] = jnp.concatenate([oa, ob], axis=1)


def _tc_call(body, grid, in_specs, out_specs, out_shape):
    return pl.pallas_call(body, grid=grid, in_specs=in_specs,
                          out_specs=out_specs, out_shape=out_shape)


def _row_spec(b, d):
    return pl.BlockSpec((b, d), lambda i: (i, 0))


def _full_spec(s0, s1):
    return pl.BlockSpec((s0, s1), lambda i: (0, 0))


def _sc_gather_body(epad, tr_hbm, tc_hbm, row_hbm, col_hbm, gr_hbm, gc_hbm,
                    idx_v, rows_v, sem):
    core = lax.axis_index("c")
    sub = lax.axis_index("s")
    wid = sub * NC + core
    nwin = epad // WIN
    k = WIN // CHUNK

    @pl.loop(0, nwin // NWORK)
    def _(i):
        win = wid + i * NWORK

        def one_side(src_idx, table, out):
            pltpu.sync_copy(src_idx.at[pl.ds(win * k, k)], idx_v)
            for half in range(2):
                cps = [pltpu.async_copy(
                    table.at[idx_v.at[half * (k // 2) + j]],
                    rows_v.at[pl.ds(j * CHUNK, CHUNK)], sem)
                    for j in range(k // 2)]
                for cp in cps:
                    cp.wait()
                pltpu.sync_copy(
                    rows_v, out.at[pl.ds(win * WIN + half * (WIN // 2),
                                         WIN // 2)])

        one_side(row_hbm, tr_hbm, gr_hbm)
        one_side(col_hbm, tc_hbm, gc_hbm)


def _sc_scatter_body(n_half, q0, gout_hbm, row_hbm, s_hbm,
                     idx_v, upd_v, acc, sem):
    core = lax.axis_index("c")
    sub = lax.axis_index("s")
    nwin = gout_hbm.shape[0] // SWIN
    k = SWIN // CHUNK
    vz = jnp.zeros((16,), F32)
    iota = lax.iota(jnp.int32, 16)

    @pl.loop(0, CHUNK)
    def _(r):
        @pl.loop(0, D // 16)
        def _(c):
            upd_v[r, pl.ds(c * 16, 16)] = vz

    for p, (poff, psize) in enumerate(((0, q0), (q0, n_half - q0))):
        base = core * n_half + poff
        nchunk = pl.cdiv(psize + CHUNK, CHUNK)

        @pl.loop(0, pl.cdiv(nchunk, NS))
        def _(i):
            c = sub + i * NS

            @pl.when(c < nchunk)
            def _():
                pltpu.sync_copy(upd_v.at[pl.ds(0, CHUNK)],
                                acc.at[pl.ds(c * CHUNK, CHUNK)])

        plsc.subcore_barrier()

        @pl.loop(0, nwin // NS)
        def _(i):
            win = sub + i * NS
            pltpu.sync_copy(row_hbm.at[pl.ds(win * k, k)], idx_v)

            @pl.loop(0, k)
            def _(j):
                @pl.loop(0, CHUNK // 16)
                def _(t):
                    v = idx_v[j, pl.ds(t * 16, 16)]
                    local = v - base
                    oob = (local < 0) | (local >= psize)
                    dump = psize + ((j * (CHUNK // 16) + t) % 5) * 16 + iota
                    idx_v[j, pl.ds(t * 16, 16)] = jnp.where(oob, dump, local)

            pltpu.sync_copy(gout_hbm.at[pl.ds(win * SWIN, SWIN)], upd_v)
            for j in range(k):
                pltpu.sync_copy(upd_v.at[pl.ds(j * CHUNK, CHUNK)],
                                acc.at[idx_v.at[j]], add=True)

        plsc.subcore_barrier()

        nfull = psize // CHUNK
        rem = psize - nfull * CHUNK

        @pl.loop(0, pl.cdiv(nfull, NS))
        def _(i):
            c = sub + i * NS

            @pl.when(c < nfull)
            def _():
                pltpu.sync_copy(acc.at[pl.ds(c * CHUNK, CHUNK)],
                                s_hbm.at[pl.ds(base + c * CHUNK, CHUNK)])

        if rem:
            @pl.when(sub == 0)
            def _():
                pltpu.sync_copy(acc.at[pl.ds(nfull * CHUNK, rem)],
                                s_hbm.at[pl.ds(base + nfull * CHUNK, rem)])

        if p == 0:
            plsc.subcore_barrier()
            @pl.loop(0, CHUNK)
            def _(r):
                @pl.loop(0, D // 16)
                def _(c):
                    upd_v[r, pl.ds(c * 16, 16)] = vz


def kernel(x, pos, vel, edge_index, edge_attr, emb_W, emb_b,
           edge_W1, edge_b1, edge_W2, edge_b2,
           node_W1, node_b1, node_W2, node_b2,
           coord_W1, coord_b1, coord_W2,
           vel_W1, vel_b1, vel_W2, vel_b2,
           head_W1, head_b1, head_W2, head_b2, head_W3, head_b3):
    n, din = x.shape
    hdim = emb_W.shape[1]
    e = edge_index.shape[1]
    nlayers = edge_W1.shape[0]
    nheads = head_W1.shape[0]

    stride = WIN * NWORK
    epad = pl.cdiv(e, stride) * stride
    n_half = pl.cdiv(n, NC)
    q0 = pl.cdiv(n_half // 2, CHUNK) * CHUNK
    acc_rows = q0 + CHUNK

    rowp = jnp.pad(edge_index[0], (0, epad - e)).reshape(-1, CHUNK)
    colp = jnp.pad(edge_index[1], (0, epad - e)).reshape(-1, CHUNK)
    eap = jnp.pad(edge_attr, ((0, epad - e), (0, 0)))
    coordp = jnp.pad(pos, ((0, 0), (0, 16 - pos.shape[1])))
    velp = jnp.pad(vel, ((0, 0), (0, 16 - vel.shape[1])))

    gn = pl.cdiv(n, BN)
    ge = epad // BE

    h = _tc_call(_emb_body, (gn,),
                 [_row_spec(BN, din), _full_spec(din, hdim),
                  _full_spec(1, hdim)],
                 _row_spec(BN, hdim),
                 jax.ShapeDtypeStruct((n, hdim), F32))(
                     x, emb_W, emb_b.reshape(1, hdim))

    mesh = plsc.VectorSubcoreMesh(core_axis_name="c", subcore_axis_name="s",
                                  num_cores=NC, num_subcores=NS)
    sc_gather = pl.kernel(
        functools.partial(_sc_gather_body, epad),
        out_type=[jax.ShapeDtypeStruct((epad, TD), F32),
                  jax.ShapeDtypeStruct((epad, TD), F32)],
        mesh=mesh,
        scratch_types=[pltpu.VMEM((WIN // CHUNK, CHUNK), jnp.int32),
                       pltpu.VMEM((WIN // 2, TD), F32),
                       pltpu.SemaphoreType.DMA])
    sc_scatter = pl.kernel(
        functools.partial(_sc_scatter_body, n_half, q0),
        out_type=jax.ShapeDtypeStruct((n, D), F32),
        mesh=mesh,
        scratch_types=[pltpu.VMEM((SWIN // CHUNK, CHUNK), jnp.int32),
                       pltpu.VMEM((SWIN, D), F32),
                       pltpu.VMEM_SHARED((acc_rows, D), F32),
                       pltpu.SemaphoreType.DMA],
        compiler_params=pltpu.CompilerParams(use_tc_tiling_on_sc=False))

    for l in range(nlayers):
        w1 = edge_W1[l]
        w1a, w1b = w1[:hdim], w1[hdim:2 * hdim]
        wr = w1[2 * hdim:2 * hdim + 1]
        w1d = w1[2 * hdim + 1:]

        tr, tc = _tc_call(
            _node_pre_body, (gn,),
            [_row_spec(BN, hdim), _row_spec(BN, 16), _full_spec(hdim, hdim),
             _full_spec(hdim, hdim), _full_spec(1, hdim)],
            [_row_spec(BN, TD), _row_spec(BN, TD)],
            [jax.ShapeDtypeStruct((n, TD), F32),
             jax.ShapeDtypeStruct((n, TD), F32)])(
                 h, coordp, w1a, w1b, edge_b1[l].reshape(1, hdim))

        gr, gc = sc_gather(tr, tc, rowp, colp)

        gout = _tc_call(
            functools.partial(_edge_body, e), (ge,),
            [_row_spec(BE, TD), _row_spec(BE, TD), _row_spec(BE, eap.shape[1]),
             _full_spec(1, hdim), _full_spec(eap.shape[1], hdim),
             _full_spec(1, hdim), _full_spec(hdim, hdim),
             _full_spec(hdim, hdim), _full_spec(1, hdim), _full_spec(1, hdim)],
            _row_spec(BE, D),
            jax.ShapeDtypeStruct((epad, D), F32))(
                gr, gc, eap, wr, w1d, edge_b2[l].reshape(1, hdim),
                edge_W2[l], coord_W1[l], coord_b1[l].reshape(1, hdim),
                coord_W2[l].reshape(1, hdim))

        s = sc_scatter(gout, rowp)

        h, coordp = _tc_call(
            _node_post_body, (gn,),
            [_row_spec(BN, D), _row_spec(BN, hdim), _row_spec(BN, 16),
             _row_spec(BN, 16), _full_spec(hdim, hdim), _full_spec(1, hdim),
             _full_spec(1, hdim), _full_spec(1, 1), _full_spec(hdim, hdim),
             _full_spec(hdim, hdim), _full_spec(1, hdim),
             _full_spec(hdim, hdim), _full_spec(1, hdim)],
            [_row_spec(BN, hdim), _row_spec(BN, 16)],
            [jax.ShapeDtypeStruct((n, hdim), F32),
             jax.ShapeDtypeStruct((n, 16), F32)])(
                s, h, coordp, velp,
                vel_W1[l], vel_b1[l].reshape(1, hdim),
                vel_W2[l].reshape(1, hdim), vel_b2[l].reshape(1, 1),
                node_W1[l][:hdim], node_W1[l][hdim:],
                node_b1[l].reshape(1, hdim), node_W2[l],
                node_b2[l].reshape(1, hdim))

    def hw(t):
        w1 = head_W1[t]
        h1 = w1[:hdim]
        c1 = jnp.pad(w1[hdim:hdim + 3], ((0, 13), (0, 0)))
        v1 = jnp.pad(w1[hdim + 3:hdim + 6], ((0, 13), (0, 0)))
        w3 = jnp.pad(head_W3[t], ((0, 0), (0, 5)))
        b3 = jnp.pad(head_b3[t], (0, 5)).reshape(1, 8)
        return (h1, c1, v1, head_b1[t].reshape(1, hdim), head_W2[t],
                head_b2[t].reshape(1, hdim), w3, b3)

    wspecs = [_full_spec(hdim, hdim), _full_spec(16, hdim),
              _full_spec(16, hdim), _full_spec(1, hdim),
              _full_spec(hdim, hdim), _full_spec(1, hdim),
              _full_spec(hdim, 8), _full_spec(1, 8)]
    out = _tc_call(
        _head_body, (gn,),
        [_row_spec(BN, hdim), _row_spec(BN, 16), _row_spec(BN, 16)]
        + wspecs + wspecs,
        _row_spec(BN, 16),
        jax.ShapeDtypeStruct((n, 16), F32))(
            h, coordp, velp, *hw(0), *hw(1))

    return out.reshape(n, nheads, 8)[:, :, :3].transpose(1, 0, 2)


# speedup vs baseline: 2.6309x; 1.1068x over previous
import functools

import jax
import jax.numpy as jnp
from jax import lax
from jax.experimental import pallas as pl
from jax.experimental.pallas import tpu as pltpu
from jax.experimental.pallas import tpu_sc as plsc

F32 = jnp.float32

TD = 128
D = 80
WIN = 1024
SWIN = 512
CHUNK = 128
NC, NS = 2, 16
NWORK = NC * NS

BN = 2000
BE = 2048


def _silu(v):
    return v * jax.nn.sigmoid(v)


def _emb_body(x_ref, w_ref, b_ref, o_ref):
    o_ref[...] = jnp.dot(x_ref[...], w_ref[...],
                         preferred_element_type=F32) + b_ref[...]


def _node_pre_body(h_ref, cp_ref, w1a_ref, w1b_ref, b1_ref, tr_ref, tc_ref):
    h = h_ref[...]
    cp = cp_ref[...]
    z = jnp.zeros((h.shape[0], TD - 80), F32)
    u = jnp.dot(h, w1a_ref[...], preferred_element_type=F32) + b1_ref[...]
    v = jnp.dot(h, w1b_ref[...], preferred_element_type=F32)
    tr_ref[...] = jnp.concatenate([u, cp, z], axis=1)
    tc_ref[...] = jnp.concatenate([v, cp, z], axis=1)


def _edge_body(n_edges, gr_ref, gc_ref, ea_ref, w132_ref, b2_ref,
               w2_ref, cw1_ref, cb1_ref, cw2p_ref, o_ref):
    gr = gr_ref[...]
    gc = gc_ref[...]
    cd = gr[:, 64:80] - gc[:, 64:80]
    cat = jnp.concatenate([cd * cd, ea_ref[...]], axis=1)
    pre = (gr[:, :64] + gc[:, :64] +
           jnp.dot(cat, w132_ref[...], preferred_element_type=F32))
    m = _silu(jnp.dot(_silu(pre), w2_ref[...],
                      preferred_element_type=F32) + b2_ref[...])
    p = _silu(jnp.dot(m, cw1_ref[...], preferred_element_type=F32) + cb1_ref[...])
    cmat = jnp.dot(p, cw2p_ref[...], preferred_element_type=F32)[:, :1]
    trans = jnp.clip(cmat * cd, -100.0, 100.0)
    lane16 = lax.broadcasted_iota(jnp.int32, (1, 16), 1)
    tpack = trans + (lane16 == 3).astype(F32)
    base = pl.program_id(0) * gr.shape[0]
    valid = (lax.broadcasted_iota(jnp.int32, (gr.shape[0], 1), 0) + base
             < n_edges).astype(F32)
    o_ref[...] = jnp.concatenate([m, tpack], axis=1) * valid


def _node_post_body(s_ref, h_ref, cp_ref, vp_ref, vw1_ref, vb1_ref, vw2_ref,
                    vb2_ref, nw1h_ref, nw1a_ref, nb1_ref, nw2_ref, nb2_ref,
                    ho_ref, co_ref):
    s = s_ref[...]
    h = h_ref[...]
    cnt = jnp.clip(s[:, 67:68], 1.0, None)
    agg = s[:, :64] / cnt
    lane16 = lax.broadcasted_iota(jnp.int32, (1, 16), 1)
    dcoord = jnp.where(lane16 < 3, s[:, 64:80], 0.0) / cnt
    sv = _silu(jnp.dot(h, vw1_ref[...], preferred_element_type=F32) + vb1_ref[...])
    vmat = jnp.sum(sv * vw2_ref[...], axis=1, keepdims=True) + vb2_ref[...]
    co_ref[...] = cp_ref[...] + dcoord + vmat * vp_ref[...]
    z = _silu(jnp.dot(h, nw1h_ref[...], preferred_element_type=F32) +
              jnp.dot(agg, nw1a_ref[...], preferred_element_type=F32) +
              nb1_ref[...])
    ho_ref[...] = h + jnp.dot(z, nw2_ref[...],
                              preferred_element_type=F32) + nb2_ref[...]


def _head_body(h_ref, cp_ref, vp_ref,
               ah1_ref, ac1_ref, av1_ref, ab1_ref, aw2_ref, ab2_ref,
               aw3_ref, ab3_ref,
               bh1_ref, bc1_ref, bv1_ref, bb1_ref, bw2_ref, bb2_ref,
               bw3_ref, bb3_ref, o_ref):
    h = h_ref[...]
    cp = cp_ref[...]
    vp = vp_ref[...]

    def head(h1, c1, v1, b1, w2, b2, w3, b3):
        z = _silu(jnp.dot(h, h1, preferred_element_type=F32) +
                  jnp.dot(cp, c1, preferred_element_type=F32) +
                  jnp.dot(vp, v1, preferred_element_type=F32) + b1)
        z = _silu(jnp.dot(z, w2, preferred_element_type=F32) + b2)
        return jnp.dot(z, w3, preferred_element_type=F32) + b3

    oa = head(ah1_ref[...], ac1_ref[...], av1_ref[...], ab1_ref[...],
              aw2_ref[...], ab2_ref[...], aw3_ref[...], ab3_ref[...])
    ob = head(bh1_ref[...], bc1_ref[...], bv1_ref[...], bb1_ref[...],
              bw2_ref[...], bb2_ref[...], bw3_ref[...], bb3_ref[...])
    o_ref[...] = jnp.concatenate([oa, ob], axis=1)


def _tc_call(body, grid, in_specs, out_specs, out_shape):
    return pl.pallas_call(body, grid=grid, in_specs=in_specs,
                          out_specs=out_specs, out_shape=out_shape)


def _row_spec(b, d):
    return pl.BlockSpec((b, d), lambda i: (i, 0))


def _full_spec(s0, s1):
    return pl.BlockSpec((s0, s1), lambda i: (0, 0))


def _sc_gather_body(epad, tr_hbm, tc_hbm, row_hbm, col_hbm, gr_hbm, gc_hbm,
                    idx_v, rows_v, sem):
    core = lax.axis_index("c")
    sub = lax.axis_index("s")
    wid = sub * NC + core
    nwin = epad // WIN
    k = WIN // CHUNK

    @pl.loop(0, nwin // NWORK)
    def _(i):
        win = wid + i * NWORK

        def one_side(src_idx, table, out):
            pltpu.sync_copy(src_idx.at[pl.ds(win * k, k)], idx_v)
            for half in range(2):
                cps = [pltpu.async_copy(
                    table.at[idx_v.at[half * (k // 2) + j]],
                    rows_v.at[pl.ds(j * CHUNK, CHUNK)], sem)
                    for j in range(k // 2)]
                for cp in cps:
                    cp.wait()
                pltpu.sync_copy(
                    rows_v, out.at[pl.ds(win * WIN + half * (WIN // 2),
                                         WIN // 2)])

        one_side(row_hbm, tr_hbm, gr_hbm)
        one_side(col_hbm, tc_hbm, gc_hbm)


def _sc_scatter_body(n_half, q0, gout_hbm, row_hbm, s_hbm,
                     idx_v, upd_v, acc, sem):
    core = lax.axis_index("c")
    sub = lax.axis_index("s")
    nwin = gout_hbm.shape[0] // SWIN
    k = SWIN // CHUNK
    vz = jnp.zeros((16,), F32)
    iota = lax.iota(jnp.int32, 16)

    @pl.loop(0, CHUNK)
    def _(r):
        @pl.loop(0, D // 16)
        def _(c):
            upd_v[r, pl.ds(c * 16, 16)] = vz

    for p, (poff, psize) in enumerate(((0, q0), (q0, n_half - q0))):
        base = core * n_half + poff
        nchunk = pl.cdiv(psize + CHUNK, CHUNK)

        @pl.loop(0, pl.cdiv(nchunk, NS))
        def _(i):
            c = sub + i * NS

            @pl.when(c < nchunk)
            def _():
                pltpu.sync_copy(upd_v.at[pl.ds(0, CHUNK)],
                                acc.at[pl.ds(c * CHUNK, CHUNK)])

        plsc.subcore_barrier()

        @pl.loop(0, nwin // NS)
        def _(i):
            win = sub + i * NS
            pltpu.sync_copy(row_hbm.at[pl.ds(win * k, k)], idx_v)

            @pl.loop(0, k)
            def _(j):
                @pl.loop(0, CHUNK // 16)
                def _(t):
                    v = idx_v[j, pl.ds(t * 16, 16)]
                    local = v - base
                    oob = (local < 0) | (local >= psize)
                    dump = psize + ((j * (CHUNK // 16) + t) % 5) * 16 + iota
                    idx_v[j, pl.ds(t * 16, 16)] = jnp.where(oob, dump, local)

            pltpu.sync_copy(gout_hbm.at[pl.ds(win * SWIN, SWIN)], upd_v)
            for j in range(k):
                pltpu.sync_copy(upd_v.at[pl.ds(j * CHUNK, CHUNK)],
                                acc.at[idx_v.at[j]], add=True)

        plsc.subcore_barrier()

        nfull = psize // CHUNK
        rem = psize - nfull * CHUNK

        @pl.loop(0, pl.cdiv(nfull, NS))
        def _(i):
            c = sub + i * NS

            @pl.when(c < nfull)
            def _():
                pltpu.sync_copy(acc.at[pl.ds(c * CHUNK, CHUNK)],
                                s_hbm.at[pl.ds(base + c * CHUNK, CHUNK)])

        if rem:
            @pl.when(sub == 0)
            def _():
                pltpu.sync_copy(acc.at[pl.ds(nfull * CHUNK, rem)],
                                s_hbm.at[pl.ds(base + nfull * CHUNK, rem)])

        if p == 0:
            plsc.subcore_barrier()
            @pl.loop(0, CHUNK)
            def _(r):
                @pl.loop(0, D // 16)
                def _(c):
                    upd_v[r, pl.ds(c * 16, 16)] = vz


def kernel(x, pos, vel, edge_index, edge_attr, emb_W, emb_b,
           edge_W1, edge_b1, edge_W2, edge_b2,
           node_W1, node_b1, node_W2, node_b2,
           coord_W1, coord_b1, coord_W2,
           vel_W1, vel_b1, vel_W2, vel_b2,
           head_W1, head_b1, head_W2, head_b2, head_W3, head_b3):
    n, din = x.shape
    hdim = emb_W.shape[1]
    e = edge_index.shape[1]
    nlayers = edge_W1.shape[0]
    nheads = head_W1.shape[0]

    stride = WIN * NWORK
    epad = pl.cdiv(e, stride) * stride
    n_half = pl.cdiv(n, NC)
    q0 = pl.cdiv(n_half // 2, CHUNK) * CHUNK
    acc_rows = q0 + CHUNK

    rowp = jnp.pad(edge_index[0], (0, epad - e)).reshape(-1, CHUNK)
    colp = jnp.pad(edge_index[1], (0, epad - e)).reshape(-1, CHUNK)
    eap = jnp.pad(edge_attr, ((0, epad - e), (0, 0)))
    coordp = jnp.pad(pos, ((0, 0), (0, 16 - pos.shape[1])))
    velp = jnp.pad(vel, ((0, 0), (0, 16 - vel.shape[1])))

    gn = pl.cdiv(n, BN)
    ge = epad // BE

    h = _tc_call(_emb_body, (gn,),
                 [_row_spec(BN, din), _full_spec(din, hdim),
                  _full_spec(1, hdim)],
                 _row_spec(BN, hdim),
                 jax.ShapeDtypeStruct((n, hdim), F32))(
                     x, emb_W, emb_b.reshape(1, hdim))

    mesh = plsc.VectorSubcoreMesh(core_axis_name="c", subcore_axis_name="s",
                                  num_cores=NC, num_subcores=NS)
    sc_gather = pl.kernel(
        functools.partial(_sc_gather_body, epad),
        out_type=[jax.ShapeDtypeStruct((epad, TD), F32),
                  jax.ShapeDtypeStruct((epad, TD), F32)],
        mesh=mesh,
        scratch_types=[pltpu.VMEM((WIN // CHUNK, CHUNK), jnp.int32),
                       pltpu.VMEM((WIN // 2, TD), F32),
                       pltpu.SemaphoreType.DMA])
    sc_scatter = pl.kernel(
        functools.partial(_sc_scatter_body, n_half, q0),
        out_type=jax.ShapeDtypeStruct((n, D), F32),
        mesh=mesh,
        scratch_types=[pltpu.VMEM((SWIN // CHUNK, CHUNK), jnp.int32),
                       pltpu.VMEM((SWIN, D), F32),
                       pltpu.VMEM_SHARED((acc_rows, D), F32),
                       pltpu.SemaphoreType.DMA],
        compiler_params=pltpu.CompilerParams(use_tc_tiling_on_sc=False))

    for l in range(nlayers):
        w1 = edge_W1[l]
        w1a, w1b = w1[:hdim], w1[hdim:2 * hdim]
        wr = w1[2 * hdim:2 * hdim + 1]
        w1d = w1[2 * hdim + 1:]
        w132 = jnp.concatenate([jnp.broadcast_to(wr, (16, hdim)), w1d], axis=0)
        cw2p = jnp.pad(coord_W2[l], ((0, 0), (0, 7)))

        tr, tc = _tc_call(
            _node_pre_body, (gn,),
            [_row_spec(BN, hdim), _row_spec(BN, 16), _full_spec(hdim, hdim),
             _full_spec(hdim, hdim), _full_spec(1, hdim)],
            [_row_spec(BN, TD), _row_spec(BN, TD)],
            [jax.ShapeDtypeStruct((n, TD), F32),
             jax.ShapeDtypeStruct((n, TD), F32)])(
                 h, coordp, w1a, w1b, edge_b1[l].reshape(1, hdim))

        gr, gc = sc_gather(tr, tc, rowp, colp)

        gout = _tc_call(
            functools.partial(_edge_body, e), (ge,),
            [_row_spec(BE, TD), _row_spec(BE, TD), _row_spec(BE, eap.shape[1]),
             _full_spec(32, hdim), _full_spec(1, hdim),
             _full_spec(hdim, hdim), _full_spec(hdim, hdim),
             _full_spec(1, hdim), _full_spec(hdim, 8)],
            _row_spec(BE, D),
            jax.ShapeDtypeStruct((epad, D), F32))(
                gr, gc, eap, w132, edge_b2[l].reshape(1, hdim),
                edge_W2[l], coord_W1[l], coord_b1[l].reshape(1, hdim),
                cw2p)

        s = sc_scatter(gout, rowp)

        h, coordp = _tc_call(
            _node_post_body, (gn,),
            [_row_spec(BN, D), _row_spec(BN, hdim), _row_spec(BN, 16),
             _row_spec(BN, 16), _full_spec(hdim, hdim), _full_spec(1, hdim),
             _full_spec(1, hdim), _full_spec(1, 1), _full_spec(hdim, hdim),
             _full_spec(hdim, hdim), _full_spec(1, hdim),
             _full_spec(hdim, hdim), _full_spec(1, hdim)],
            [_row_spec(BN, hdim), _row_spec(BN, 16)],
            [jax.ShapeDtypeStruct((n, hdim), F32),
             jax.ShapeDtypeStruct((n, 16), F32)])(
                s, h, coordp, velp,
                vel_W1[l], vel_b1[l].reshape(1, hdim),
                vel_W2[l].reshape(1, hdim), vel_b2[l].reshape(1, 1),
                node_W1[l][:hdim], node_W1[l][hdim:],
                node_b1[l].reshape(1, hdim), node_W2[l],
                node_b2[l].reshape(1, hdim))

    def hw(t):
        w1 = head_W1[t]
        h1 = w1[:hdim]
        c1 = jnp.pad(w1[hdim:hdim + 3], ((0, 13), (0, 0)))
        v1 = jnp.pad(w1[hdim + 3:hdim + 6], ((0, 13), (0, 0)))
        w3 = jnp.pad(head_W3[t], ((0, 0), (0, 5)))
        b3 = jnp.pad(head_b3[t], (0, 5)).reshape(1, 8)
        return (h1, c1, v1, head_b1[t].reshape(1, hdim), head_W2[t],
                head_b2[t].reshape(1, hdim), w3, b3)

    wspecs = [_full_spec(hdim, hdim), _full_spec(16, hdim),
              _full_spec(16, hdim), _full_spec(1, hdim),
              _full_spec(hdim, hdim), _full_spec(1, hdim),
              _full_spec(hdim, 8), _full_spec(1, 8)]
    out = _tc_call(
        _head_body, (gn,),
        [_row_spec(BN, hdim), _row_spec(BN, 16), _row_spec(BN, 16)]
        + wspecs + wspecs,
        _row_spec(BN, 16),
        jax.ShapeDtypeStruct((n, 16), F32))(
            h, coordp, velp, *hw(0), *hw(1))

    return out.reshape(n, nheads, 8)[:, :, :3].transpose(1, 0, 2)


# speedup vs baseline: 2.6855x; 1.0208x over previous
import functools

import jax
import jax.numpy as jnp
from jax import lax
from jax.experimental import pallas as pl
from jax.experimental.pallas import tpu as pltpu
from jax.experimental.pallas import tpu_sc as plsc

F32 = jnp.float32

TD = 128
D = 80
WIN = 1024
SWIN = 512
CHUNK = 128
NC, NS = 2, 16
NWORK = NC * NS

BN = 2000
BE = 2048


def _silu(v):
    return v * jax.nn.sigmoid(v)


def _emb_body(x_ref, w_ref, b_ref, o_ref):
    o_ref[...] = jnp.dot(x_ref[...], w_ref[...],
                         preferred_element_type=F32) + b_ref[...]


def _node_pre_body(h_ref, cp_ref, w1a_ref, w1b_ref, b1_ref, tr_ref, tc_ref):
    h = h_ref[...]
    cp = cp_ref[...]
    z = jnp.zeros((h.shape[0], TD - 80), F32)
    u = jnp.dot(h, w1a_ref[...], preferred_element_type=F32) + b1_ref[...]
    v = jnp.dot(h, w1b_ref[...], preferred_element_type=F32)
    tr_ref[...] = jnp.concatenate([u, cp, z], axis=1)
    tc_ref[...] = jnp.concatenate([v, cp, z], axis=1)


def _edge_body(n_edges, gr_ref, gc_ref, ea_ref, w132_ref, b2_ref,
               w2_ref, cw1_ref, cb1_ref, cw2p_ref, o_ref):
    gr = gr_ref[...]
    gc = gc_ref[...]
    cd = gr[:, 64:80] - gc[:, 64:80]
    cat = jnp.concatenate([cd * cd, ea_ref[...]], axis=1)
    pre = (gr[:, :64] + gc[:, :64] +
           jnp.dot(cat, w132_ref[...], preferred_element_type=F32))
    m = _silu(jnp.dot(_silu(pre), w2_ref[...],
                      preferred_element_type=F32) + b2_ref[...])
    p = _silu(jnp.dot(m, cw1_ref[...], preferred_element_type=F32) + cb1_ref[...])
    cmat = jnp.dot(p, cw2p_ref[...], preferred_element_type=F32)[:, :1]
    trans = jnp.clip(cmat * cd, -100.0, 100.0)
    lane16 = lax.broadcasted_iota(jnp.int32, (1, 16), 1)
    tpack = trans + (lane16 == 3).astype(F32)
    base = pl.program_id(0) * gr.shape[0]
    valid = (lax.broadcasted_iota(jnp.int32, (gr.shape[0], 1), 0) + base
             < n_edges).astype(F32)
    o_ref[...] = jnp.concatenate([m, tpack], axis=1) * valid


def _node_post_body(s_ref, h_ref, cp_ref, vp_ref, vw1_ref, vb1_ref, vw2_ref,
                    vb2_ref, nw1h_ref, nw1a_ref, nb1_ref, nw2_ref, nb2_ref,
                    ho_ref, co_ref):
    s = s_ref[...]
    h = h_ref[...]
    cnt = jnp.clip(s[:, 67:68], 1.0, None)
    agg = s[:, :64] / cnt
    lane16 = lax.broadcasted_iota(jnp.int32, (1, 16), 1)
    dcoord = jnp.where(lane16 < 3, s[:, 64:80], 0.0) / cnt
    sv = _silu(jnp.dot(h, vw1_ref[...], preferred_element_type=F32) + vb1_ref[...])
    vmat = jnp.sum(sv * vw2_ref[...], axis=1, keepdims=True) + vb2_ref[...]
    co_ref[...] = cp_ref[...] + dcoord + vmat * vp_ref[...]
    z = _silu(jnp.dot(h, nw1h_ref[...], preferred_element_type=F32) +
              jnp.dot(agg, nw1a_ref[...], preferred_element_type=F32) +
              nb1_ref[...])
    ho_ref[...] = h + jnp.dot(z, nw2_ref[...],
                              preferred_element_type=F32) + nb2_ref[...]


def _head_body(h_ref, cp_ref, vp_ref,
               ah1_ref, ac1_ref, av1_ref, ab1_ref, aw2_ref, ab2_ref,
               aw3_ref, ab3_ref,
               bh1_ref, bc1_ref, bv1_ref, bb1_ref, bw2_ref, bb2_ref,
               bw3_ref, bb3_ref, o_ref):
    h = h_ref[...]
    cp = cp_ref[...]
    vp = vp_ref[...]

    def head(h1, c1, v1, b1, w2, b2, w3, b3):
        z = _silu(jnp.dot(h, h1, preferred_element_type=F32) +
                  jnp.dot(cp, c1, preferred_element_type=F32) +
                  jnp.dot(vp, v1, preferred_element_type=F32) + b1)
        z = _silu(jnp.dot(z, w2, preferred_element_type=F32) + b2)
        return jnp.dot(z, w3, preferred_element_type=F32) + b3

    oa = head(ah1_ref[...], ac1_ref[...], av1_ref[...], ab1_ref[...],
              aw2_ref[...], ab2_ref[...], aw3_ref[...], ab3_ref[...])
    ob = head(bh1_ref[...], bc1_ref[...], bv1_ref[...], bb1_ref[...],
              bw2_ref[...], bb2_ref[...], bw3_ref[...], bb3_ref[...])
    o_ref[...] = jnp.concatenate([oa, ob], axis=1)


def _tc_call(body, grid, in_specs, out_specs, out_shape):
    return pl.pallas_call(body, grid=grid, in_specs=in_specs,
                          out_specs=out_specs, out_shape=out_shape)


def _row_spec(b, d):
    return pl.BlockSpec((b, d), lambda i: (i, 0))


def _full_spec(s0, s1):
    return pl.BlockSpec((s0, s1), lambda i: (0, 0))


def _sc_gather_body(epad, tr_hbm, tc_hbm, row_hbm, col_hbm, gr_hbm, gc_hbm,
                    idxr_v, idxc_v, bufa, bufb, sga, sgb, swa, swb):
    core = lax.axis_index("c")
    sub = lax.axis_index("s")
    wid = sub * NC + core
    nwin = epad // WIN
    k = WIN // CHUNK
    sc_rows = 2 * CHUNK
    bufs = (bufa, bufb)
    gsems = (sga, sgb)
    wsems = (swa, swb)

    @pl.loop(0, nwin // NWORK)
    def _(i):
        win = wid + i * NWORK
        pltpu.sync_copy(row_hbm.at[pl.ds(win * k, k)], idxr_v)
        pltpu.sync_copy(col_hbm.at[pl.ds(win * k, k)], idxc_v)

        sides = ((idxr_v, tr_hbm, gr_hbm), (idxc_v, tc_hbm, gc_hbm))
        prev = None
        pend_w = [None, None]
        for s in range(2 * (k // 2)):
            idx_v, table, out = sides[s // (k // 2)]
            j0 = (s % (k // 2)) * 2
            b = s % 2
            if pend_w[b] is not None:
                pend_w[b].wait()
                pend_w[b] = None
            gh = [pltpu.async_copy(table.at[idx_v.at[j0 + t]],
                                   bufs[b].at[pl.ds(t * CHUNK, CHUNK)],
                                   gsems[b])
                  for t in range(2)]
            if prev is not None:
                ps, pgh = prev
                for h in pgh:
                    h.wait()
                pb = ps % 2
                off = win * WIN + (ps % (k // 2)) * sc_rows
                pout = sides[ps // (k // 2)][2]
                pend_w[pb] = pltpu.async_copy(
                    bufs[pb], pout.at[pl.ds(off, sc_rows)], wsems[pb])
            prev = (s, gh)
        ps, pgh = prev
        for h in pgh:
            h.wait()
        pb = ps % 2
        off = win * WIN + (ps % (k // 2)) * sc_rows
        pend_w[pb] = pltpu.async_copy(
            bufs[pb], sides[ps // (k // 2)][2].at[pl.ds(off, sc_rows)],
            wsems[pb])
        for b in range(2):
            if pend_w[b] is not None:
                pend_w[b].wait()


def _sc_scatter_body(n_half, q0, gout_hbm, row_hbm, s_hbm,
                     idx_v, upd_v, acc, sem):
    core = lax.axis_index("c")
    sub = lax.axis_index("s")
    nwin = gout_hbm.shape[0] // SWIN
    k = SWIN // CHUNK
    vz = jnp.zeros((16,), F32)
    iota = lax.iota(jnp.int32, 16)

    @pl.loop(0, CHUNK)
    def _(r):
        @pl.loop(0, D // 16)
        def _(c):
            upd_v[r, pl.ds(c * 16, 16)] = vz

    for p, (poff, psize) in enumerate(((0, q0), (q0, n_half - q0))):
        base = core * n_half + poff
        nchunk = pl.cdiv(psize + CHUNK, CHUNK)

        @pl.loop(0, pl.cdiv(nchunk, NS))
        def _(i):
            c = sub + i * NS

            @pl.when(c < nchunk)
            def _():
                pltpu.sync_copy(upd_v.at[pl.ds(0, CHUNK)],
                                acc.at[pl.ds(c * CHUNK, CHUNK)])

        plsc.subcore_barrier()

        @pl.loop(0, nwin // NS)
        def _(i):
            win = sub + i * NS
            pltpu.sync_copy(row_hbm.at[pl.ds(win * k, k)], idx_v)

            @pl.loop(0, k)
            def _(j):
                @pl.loop(0, CHUNK // 16)
                def _(t):
                    v = idx_v[j, pl.ds(t * 16, 16)]
                    local = v - base
                    oob = (local < 0) | (local >= psize)
                    dump = psize + ((j * (CHUNK // 16) + t) % 5) * 16 + iota
                    idx_v[j, pl.ds(t * 16, 16)] = jnp.where(oob, dump, local)

            pltpu.sync_copy(gout_hbm.at[pl.ds(win * SWIN, SWIN)], upd_v)
            for j in range(k):
                pltpu.sync_copy(upd_v.at[pl.ds(j * CHUNK, CHUNK)],
                                acc.at[idx_v.at[j]], add=True)

        plsc.subcore_barrier()

        nfull = psize // CHUNK
        rem = psize - nfull * CHUNK

        @pl.loop(0, pl.cdiv(nfull, NS))
        def _(i):
            c = sub + i * NS

            @pl.when(c < nfull)
            def _():
                pltpu.sync_copy(acc.at[pl.ds(c * CHUNK, CHUNK)],
                                s_hbm.at[pl.ds(base + c * CHUNK, CHUNK)])

        if rem:
            @pl.when(sub == 0)
            def _():
                pltpu.sync_copy(acc.at[pl.ds(nfull * CHUNK, rem)],
                                s_hbm.at[pl.ds(base + nfull * CHUNK, rem)])

        if p == 0:
            plsc.subcore_barrier()
            @pl.loop(0, CHUNK)
            def _(r):
                @pl.loop(0, D // 16)
                def _(c):
                    upd_v[r, pl.ds(c * 16, 16)] = vz


def kernel(x, pos, vel, edge_index, edge_attr, emb_W, emb_b,
           edge_W1, edge_b1, edge_W2, edge_b2,
           node_W1, node_b1, node_W2, node_b2,
           coord_W1, coord_b1, coord_W2,
           vel_W1, vel_b1, vel_W2, vel_b2,
           head_W1, head_b1, head_W2, head_b2, head_W3, head_b3):
    n, din = x.shape
    hdim = emb_W.shape[1]
    e = edge_index.shape[1]
    nlayers = edge_W1.shape[0]
    nheads = head_W1.shape[0]

    stride = WIN * NWORK
    epad = pl.cdiv(e, stride) * stride
    n_half = pl.cdiv(n, NC)
    q0 = pl.cdiv(n_half // 2, CHUNK) * CHUNK
    acc_rows = q0 + CHUNK

    rowp = jnp.pad(edge_index[0], (0, epad - e)).reshape(-1, CHUNK)
    colp = jnp.pad(edge_index[1], (0, epad - e)).reshape(-1, CHUNK)
    eap = jnp.pad(edge_attr, ((0, epad - e), (0, 0)))
    coordp = jnp.pad(pos, ((0, 0), (0, 16 - pos.shape[1])))
    velp = jnp.pad(vel, ((0, 0), (0, 16 - vel.shape[1])))

    gn = pl.cdiv(n, BN)
    ge = epad // BE

    h = _tc_call(_emb_body, (gn,),
                 [_row_spec(BN, din), _full_spec(din, hdim),
                  _full_spec(1, hdim)],
                 _row_spec(BN, hdim),
                 jax.ShapeDtypeStruct((n, hdim), F32))(
                     x, emb_W, emb_b.reshape(1, hdim))

    mesh = plsc.VectorSubcoreMesh(core_axis_name="c", subcore_axis_name="s",
                                  num_cores=NC, num_subcores=NS)
    sc_gather = pl.kernel(
        functools.partial(_sc_gather_body, epad),
        out_type=[jax.ShapeDtypeStruct((epad, TD), F32),
                  jax.ShapeDtypeStruct((epad, TD), F32)],
        mesh=mesh,
        scratch_types=[pltpu.VMEM((WIN // CHUNK, CHUNK), jnp.int32),
                       pltpu.VMEM((WIN // CHUNK, CHUNK), jnp.int32),
                       pltpu.VMEM((2 * CHUNK, TD), F32),
                       pltpu.VMEM((2 * CHUNK, TD), F32),
                       pltpu.SemaphoreType.DMA, pltpu.SemaphoreType.DMA,
                       pltpu.SemaphoreType.DMA, pltpu.SemaphoreType.DMA])
    sc_scatter = pl.kernel(
        functools.partial(_sc_scatter_body, n_half, q0),
        out_type=jax.ShapeDtypeStruct((n, D), F32),
        mesh=mesh,
        scratch_types=[pltpu.VMEM((SWIN // CHUNK, CHUNK), jnp.int32),
                       pltpu.VMEM((SWIN, D), F32),
                       pltpu.VMEM_SHARED((acc_rows, D), F32),
                       pltpu.SemaphoreType.DMA],
        compiler_params=pltpu.CompilerParams(use_tc_tiling_on_sc=False))

    for l in range(nlayers):
        w1 = edge_W1[l]
        w1a, w1b = w1[:hdim], w1[hdim:2 * hdim]
        wr = w1[2 * hdim:2 * hdim + 1]
        w1d = w1[2 * hdim + 1:]
        w132 = jnp.concatenate([jnp.broadcast_to(wr, (16, hdim)), w1d], axis=0)
        cw2p = jnp.pad(coord_W2[l], ((0, 0), (0, 7)))

        tr, tc = _tc_call(
            _node_pre_body, (gn,),
            [_row_spec(BN, hdim), _row_spec(BN, 16), _full_spec(hdim, hdim),
             _full_spec(hdim, hdim), _full_spec(1, hdim)],
            [_row_spec(BN, TD), _row_spec(BN, TD)],
            [jax.ShapeDtypeStruct((n, TD), F32),
             jax.ShapeDtypeStruct((n, TD), F32)])(
                 h, coordp, w1a, w1b, edge_b1[l].reshape(1, hdim))

        gr, gc = sc_gather(tr, tc, rowp, colp)

        gout = _tc_call(
            functools.partial(_edge_body, e), (ge,),
            [_row_spec(BE, TD), _row_spec(BE, TD), _row_spec(BE, eap.shape[1]),
             _full_spec(32, hdim), _full_spec(1, hdim),
             _full_spec(hdim, hdim), _full_spec(hdim, hdim),
             _full_spec(1, hdim), _full_spec(hdim, 8)],
            _row_spec(BE, D),
            jax.ShapeDtypeStruct((epad, D), F32))(
                gr, gc, eap, w132, edge_b2[l].reshape(1, hdim),
                edge_W2[l], coord_W1[l], coord_b1[l].reshape(1, hdim),
                cw2p)

        s = sc_scatter(gout, rowp)

        h, coordp = _tc_call(
            _node_post_body, (gn,),
            [_row_spec(BN, D), _row_spec(BN, hdim), _row_spec(BN, 16),
             _row_spec(BN, 16), _full_spec(hdim, hdim), _full_spec(1, hdim),
             _full_spec(1, hdim), _full_spec(1, 1), _full_spec(hdim, hdim),
             _full_spec(hdim, hdim), _full_spec(1, hdim),
             _full_spec(hdim, hdim), _full_spec(1, hdim)],
            [_row_spec(BN, hdim), _row_spec(BN, 16)],
            [jax.ShapeDtypeStruct((n, hdim), F32),
             jax.ShapeDtypeStruct((n, 16), F32)])(
                s, h, coordp, velp,
                vel_W1[l], vel_b1[l].reshape(1, hdim),
                vel_W2[l].reshape(1, hdim), vel_b2[l].reshape(1, 1),
                node_W1[l][:hdim], node_W1[l][hdim:],
                node_b1[l].reshape(1, hdim), node_W2[l],
                node_b2[l].reshape(1, hdim))

    def hw(t):
        w1 = head_W1[t]
        h1 = w1[:hdim]
        c1 = jnp.pad(w1[hdim:hdim + 3], ((0, 13), (0, 0)))
        v1 = jnp.pad(w1[hdim + 3:hdim + 6], ((0, 13), (0, 0)))
        w3 = jnp.pad(head_W3[t], ((0, 0), (0, 5)))
        b3 = jnp.pad(head_b3[t], (0, 5)).reshape(1, 8)
        return (h1, c1, v1, head_b1[t].reshape(1, hdim), head_W2[t],
                head_b2[t].reshape(1, hdim), w3, b3)

    wspecs = [_full_spec(hdim, hdim), _full_spec(16, hdim),
              _full_spec(16, hdim), _full_spec(1, hdim),
              _full_spec(hdim, hdim), _full_spec(1, hdim),
              _full_spec(hdim, 8), _full_spec(1, 8)]
    out = _tc_call(
        _head_body, (gn,),
        [_row_spec(BN, hdim), _row_spec(BN, 16), _row_spec(BN, 16)]
        + wspecs + wspecs,
        _row_spec(BN, 16),
        jax.ShapeDtypeStruct((n, 16), F32))(
            h, coordp, velp, *hw(0), *hw(1))

    return out.reshape(n, nheads, 8)[:, :, :3].transpose(1, 0, 2)


# speedup vs baseline: 2.7487x; 1.0235x over previous
import functools

import jax
import jax.numpy as jnp
from jax import lax
from jax.experimental import pallas as pl
from jax.experimental.pallas import tpu as pltpu
from jax.experimental.pallas import tpu_sc as plsc

F32 = jnp.float32

TD = 128
D = 80
WIN = 1024
SWIN = 256
CHUNK = 128
NC, NS = 2, 16
NWORK = NC * NS

BN = 2000
BE = 2048


def _silu(v):
    return v * jax.nn.sigmoid(v)


def _emb_body(x_ref, w_ref, b_ref, o_ref):
    o_ref[...] = jnp.dot(x_ref[...], w_ref[...],
                         preferred_element_type=F32) + b_ref[...]


def _node_pre_body(h_ref, cp_ref, w1a_ref, w1b_ref, b1_ref, tr_ref, tc_ref):
    h = h_ref[...]
    cp = cp_ref[...]
    z = jnp.zeros((h.shape[0], TD - 80), F32)
    u = jnp.dot(h, w1a_ref[...], preferred_element_type=F32) + b1_ref[...]
    v = jnp.dot(h, w1b_ref[...], preferred_element_type=F32)
    tr_ref[...] = jnp.concatenate([u, cp, z], axis=1)
    tc_ref[...] = jnp.concatenate([v, cp, z], axis=1)


def _edge_body(n_edges, gr_ref, gc_ref, ea_ref, w132_ref, b2_ref,
               w2_ref, cw1_ref, cb1_ref, cw2p_ref, o_ref):
    gr = gr_ref[...]
    gc = gc_ref[...]
    cd = gr[:, 64:80] - gc[:, 64:80]
    cat = jnp.concatenate([cd * cd, ea_ref[...]], axis=1)
    pre = (gr[:, :64] + gc[:, :64] +
           jnp.dot(cat, w132_ref[...], preferred_element_type=F32))
    m = _silu(jnp.dot(_silu(pre), w2_ref[...],
                      preferred_element_type=F32) + b2_ref[...])
    p = _silu(jnp.dot(m, cw1_ref[...], preferred_element_type=F32) + cb1_ref[...])
    cmat = jnp.dot(p, cw2p_ref[...], preferred_element_type=F32)[:, :1]
    trans = jnp.clip(cmat * cd, -100.0, 100.0)
    lane16 = lax.broadcasted_iota(jnp.int32, (1, 16), 1)
    tpack = trans + (lane16 == 3).astype(F32)
    base = pl.program_id(0) * gr.shape[0]
    valid = (lax.broadcasted_iota(jnp.int32, (gr.shape[0], 1), 0) + base
             < n_edges).astype(F32)
    o_ref[...] = jnp.concatenate([m, tpack], axis=1) * valid


def _node_post_body(s_ref, h_ref, cp_ref, vp_ref, vw1_ref, vb1_ref, vw2_ref,
                    vb2_ref, nw1h_ref, nw1a_ref, nb1_ref, nw2_ref, nb2_ref,
                    ho_ref, co_ref):
    s = s_ref[...]
    h = h_ref[...]
    cnt = jnp.clip(s[:, 67:68], 1.0, None)
    agg = s[:, :64] / cnt
    lane16 = lax.broadcasted_iota(jnp.int32, (1, 16), 1)
    dcoord = jnp.where(lane16 < 3, s[:, 64:80], 0.0) / cnt
    sv = _silu(jnp.dot(h, vw1_ref[...], preferred_element_type=F32) + vb1_ref[...])
    vmat = jnp.sum(sv * vw2_ref[...], axis=1, keepdims=True) + vb2_ref[...]
    co_ref[...] = cp_ref[...] + dcoord + vmat * vp_ref[...]
    z = _silu(jnp.dot(h, nw1h_ref[...], preferred_element_type=F32) +
              jnp.dot(agg, nw1a_ref[...], preferred_element_type=F32) +
              nb1_ref[...])
    ho_ref[...] = h + jnp.dot(z, nw2_ref[...],
                              preferred_element_type=F32) + nb2_ref[...]


def _head_body(h_ref, cp_ref, vp_ref,
               ah1_ref, ac1_ref, av1_ref, ab1_ref, aw2_ref, ab2_ref,
               aw3_ref, ab3_ref,
               bh1_ref, bc1_ref, bv1_ref, bb1_ref, bw2_ref, bb2_ref,
               bw3_ref, bb3_ref, o_ref):
    h = h_ref[...]
    cp = cp_ref[...]
    vp = vp_ref[...]

    def head(h1, c1, v1, b1, w2, b2, w3, b3):
        z = _silu(jnp.dot(h, h1, preferred_element_type=F32) +
                  jnp.dot(cp, c1, preferred_element_type=F32) +
                  jnp.dot(vp, v1, preferred_element_type=F32) + b1)
        z = _silu(jnp.dot(z, w2, preferred_element_type=F32) + b2)
        return jnp.dot(z, w3, preferred_element_type=F32) + b3

    oa = head(ah1_ref[...], ac1_ref[...], av1_ref[...], ab1_ref[...],
              aw2_ref[...], ab2_ref[...], aw3_ref[...], ab3_ref[...])
    ob = head(bh1_ref[...], bc1_ref[...], bv1_ref[...], bb1_ref[...],
              bw2_ref[...], bb2_ref[...], bw3_ref[...], bb3_ref[...])
    o_ref[...] = jnp.concatenate([oa, ob], axis=1)


def _tc_call(body, grid, in_specs, out_specs, out_shape):
    return pl.pallas_call(body, grid=grid, in_specs=in_specs,
                          out_specs=out_specs, out_shape=out_shape)


def _row_spec(b, d):
    return pl.BlockSpec((b, d), lambda i: (i, 0))


def _full_spec(s0, s1):
    return pl.BlockSpec((s0, s1), lambda i: (0, 0))


def _sc_gather_body(epad, tr_hbm, tc_hbm, row_hbm, col_hbm, gr_hbm, gc_hbm,
                    idxr_v, idxc_v, bufa, bufb, sga, sgb, swa, swb):
    core = lax.axis_index("c")
    sub = lax.axis_index("s")
    wid = sub * NC + core
    nwin = epad // WIN
    k = WIN // CHUNK
    sc_rows = 2 * CHUNK
    bufs = (bufa, bufb)
    gsems = (sga, sgb)
    wsems = (swa, swb)

    @pl.loop(0, nwin // NWORK)
    def _(i):
        win = wid + i * NWORK
        pltpu.sync_copy(row_hbm.at[pl.ds(win * k, k)], idxr_v)
        pltpu.sync_copy(col_hbm.at[pl.ds(win * k, k)], idxc_v)

        sides = ((idxr_v, tr_hbm, gr_hbm), (idxc_v, tc_hbm, gc_hbm))
        prev = None
        pend_w = [None, None]
        for s in range(2 * (k // 2)):
            idx_v, table, out = sides[s // (k // 2)]
            j0 = (s % (k // 2)) * 2
            b = s % 2
            if pend_w[b] is not None:
                pend_w[b].wait()
                pend_w[b] = None
            gh = [pltpu.async_copy(table.at[idx_v.at[j0 + t]],
                                   bufs[b].at[pl.ds(t * CHUNK, CHUNK)],
                                   gsems[b])
                  for t in range(2)]
            if prev is not None:
                ps, pgh = prev
                for h in pgh:
                    h.wait()
                pb = ps % 2
                off = win * WIN + (ps % (k // 2)) * sc_rows
                pout = sides[ps // (k // 2)][2]
                pend_w[pb] = pltpu.async_copy(
                    bufs[pb], pout.at[pl.ds(off, sc_rows)], wsems[pb])
            prev = (s, gh)
        ps, pgh = prev
        for h in pgh:
            h.wait()
        pb = ps % 2
        off = win * WIN + (ps % (k // 2)) * sc_rows
        pend_w[pb] = pltpu.async_copy(
            bufs[pb], sides[ps // (k // 2)][2].at[pl.ds(off, sc_rows)],
            wsems[pb])
        for b in range(2):
            if pend_w[b] is not None:
                pend_w[b].wait()


def _sc_scatter_body(n_half, q0, gout_hbm, row_hbm, s_hbm,
                     idxa_v, idxb_v, bufa, bufb, acc, sua, sub_sem, ssa, ssb):
    core = lax.axis_index("c")
    sub = lax.axis_index("s")
    nwin = gout_hbm.shape[0] // SWIN
    k = SWIN // CHUNK
    vz = jnp.zeros((16,), F32)
    iota = lax.iota(jnp.int32, 16)

    def zero_buf():
        @pl.loop(0, CHUNK)
        def _(r):
            @pl.loop(0, D // 16)
            def _(c):
                bufa[r, pl.ds(c * 16, 16)] = vz

    zero_buf()

    for p, (poff, psize) in enumerate(((0, q0), (q0, n_half - q0))):
        base = core * n_half + poff
        nchunk = pl.cdiv(psize + CHUNK, CHUNK)

        @pl.loop(0, pl.cdiv(nchunk, NS))
        def _(i):
            c = sub + i * NS

            @pl.when(c < nchunk)
            def _():
                pltpu.sync_copy(bufa.at[pl.ds(0, CHUNK)],
                                acc.at[pl.ds(c * CHUNK, CHUNK)])

        plsc.subcore_barrier()

        def remap(idx_v):
            @pl.loop(0, k)
            def _(j):
                @pl.loop(0, CHUNK // 16)
                def _(t):
                    v = idx_v[j, pl.ds(t * 16, 16)]
                    local = v - base
                    oob = (local < 0) | (local >= psize)
                    dump = psize + ((j * (CHUNK // 16) + t) % 5) * 16 + iota
                    idx_v[j, pl.ds(t * 16, 16)] = jnp.where(oob, dump, local)

        @pl.loop(0, nwin // NS // 2)
        def _(i):
            w0 = sub + (2 * i) * NS
            w1 = sub + (2 * i + 1) * NS
            pltpu.sync_copy(row_hbm.at[pl.ds(w0 * k, k)], idxa_v)
            ha = pltpu.async_copy(gout_hbm.at[pl.ds(w0 * SWIN, SWIN)],
                                  bufa, sua)
            pltpu.sync_copy(row_hbm.at[pl.ds(w1 * k, k)], idxb_v)
            hb = pltpu.async_copy(gout_hbm.at[pl.ds(w1 * SWIN, SWIN)],
                                  bufb, sub_sem)
            remap(idxa_v)
            remap(idxb_v)
            ha.wait()
            adds_a = [pltpu.async_copy(bufa.at[pl.ds(j * CHUNK, CHUNK)],
                                       acc.at[idxa_v.at[j]], ssa, add=True)
                      for j in range(k)]
            hb.wait()
            adds_b = [pltpu.async_copy(bufb.at[pl.ds(j * CHUNK, CHUNK)],
                                       acc.at[idxb_v.at[j]], ssb, add=True)
                      for j in range(k)]
            for h in adds_a + adds_b:
                h.wait()

        plsc.subcore_barrier()

        nfull = psize // CHUNK
        rem = psize - nfull * CHUNK

        @pl.loop(0, pl.cdiv(nfull, NS))
        def _(i):
            c = sub + i * NS

            @pl.when(c < nfull)
            def _():
                pltpu.sync_copy(acc.at[pl.ds(c * CHUNK, CHUNK)],
                                s_hbm.at[pl.ds(base + c * CHUNK, CHUNK)])

        if rem:
            @pl.when(sub == 0)
            def _():
                pltpu.sync_copy(acc.at[pl.ds(nfull * CHUNK, rem)],
                                s_hbm.at[pl.ds(base + nfull * CHUNK, rem)])

        if p == 0:
            plsc.subcore_barrier()
            zero_buf()


def kernel(x, pos, vel, edge_index, edge_attr, emb_W, emb_b,
           edge_W1, edge_b1, edge_W2, edge_b2,
           node_W1, node_b1, node_W2, node_b2,
           coord_W1, coord_b1, coord_W2,
           vel_W1, vel_b1, vel_W2, vel_b2,
           head_W1, head_b1, head_W2, head_b2, head_W3, head_b3):
    n, din = x.shape
    hdim = emb_W.shape[1]
    e = edge_index.shape[1]
    nlayers = edge_W1.shape[0]
    nheads = head_W1.shape[0]

    stride = WIN * NWORK
    epad = pl.cdiv(e, stride) * stride
    n_half = pl.cdiv(n, NC)
    q0 = pl.cdiv(n_half // 2, CHUNK) * CHUNK
    acc_rows = q0 + CHUNK

    rowp = jnp.pad(edge_index[0], (0, epad - e)).reshape(-1, CHUNK)
    colp = jnp.pad(edge_index[1], (0, epad - e)).reshape(-1, CHUNK)
    eap = jnp.pad(edge_attr, ((0, epad - e), (0, 0)))
    coordp = jnp.pad(pos, ((0, 0), (0, 16 - pos.shape[1])))
    velp = jnp.pad(vel, ((0, 0), (0, 16 - vel.shape[1])))

    gn = pl.cdiv(n, BN)
    ge = epad // BE

    h = _tc_call(_emb_body, (gn,),
                 [_row_spec(BN, din), _full_spec(din, hdim),
                  _full_spec(1, hdim)],
                 _row_spec(BN, hdim),
                 jax.ShapeDtypeStruct((n, hdim), F32))(
                     x, emb_W, emb_b.reshape(1, hdim))

    mesh = plsc.VectorSubcoreMesh(core_axis_name="c", subcore_axis_name="s",
                                  num_cores=NC, num_subcores=NS)
    sc_gather = pl.kernel(
        functools.partial(_sc_gather_body, epad),
        out_type=[jax.ShapeDtypeStruct((epad, TD), F32),
                  jax.ShapeDtypeStruct((epad, TD), F32)],
        mesh=mesh,
        scratch_types=[pltpu.VMEM((WIN // CHUNK, CHUNK), jnp.int32),
                       pltpu.VMEM((WIN // CHUNK, CHUNK), jnp.int32),
                       pltpu.VMEM((2 * CHUNK, TD), F32),
                       pltpu.VMEM((2 * CHUNK, TD), F32),
                       pltpu.SemaphoreType.DMA, pltpu.SemaphoreType.DMA,
                       pltpu.SemaphoreType.DMA, pltpu.SemaphoreType.DMA])
    sc_scatter = pl.kernel(
        functools.partial(_sc_scatter_body, n_half, q0),
        out_type=jax.ShapeDtypeStruct((n, D), F32),
        mesh=mesh,
        scratch_types=[pltpu.VMEM((SWIN // CHUNK, CHUNK), jnp.int32),
                       pltpu.VMEM((SWIN // CHUNK, CHUNK), jnp.int32),
                       pltpu.VMEM((SWIN, D), F32),
                       pltpu.VMEM((SWIN, D), F32),
                       pltpu.VMEM_SHARED((acc_rows, D), F32),
                       pltpu.SemaphoreType.DMA, pltpu.SemaphoreType.DMA,
                       pltpu.SemaphoreType.DMA, pltpu.SemaphoreType.DMA],
        compiler_params=pltpu.CompilerParams(use_tc_tiling_on_sc=False))

    for l in range(nlayers):
        w1 = edge_W1[l]
        w1a, w1b = w1[:hdim], w1[hdim:2 * hdim]
        wr = w1[2 * hdim:2 * hdim + 1]
        w1d = w1[2 * hdim + 1:]
        w132 = jnp.concatenate([jnp.broadcast_to(wr, (16, hdim)), w1d], axis=0)
        cw2p = jnp.pad(coord_W2[l], ((0, 0), (0, 7)))

        tr, tc = _tc_call(
            _node_pre_body, (gn,),
            [_row_spec(BN, hdim), _row_spec(BN, 16), _full_spec(hdim, hdim),
             _full_spec(hdim, hdim), _full_spec(1, hdim)],
            [_row_spec(BN, TD), _row_spec(BN, TD)],
            [jax.ShapeDtypeStruct((n, TD), F32),
             jax.ShapeDtypeStruct((n, TD), F32)])(
                 h, coordp, w1a, w1b, edge_b1[l].reshape(1, hdim))

        gr, gc = sc_gather(tr, tc, rowp, colp)

        gout = _tc_call(
            functools.partial(_edge_body, e), (ge,),
            [_row_spec(BE, TD), _row_spec(BE, TD), _row_spec(BE, eap.shape[1]),
             _full_spec(32, hdim), _full_spec(1, hdim),
             _full_spec(hdim, hdim), _full_spec(hdim, hdim),
             _full_spec(1, hdim), _full_spec(hdim, 8)],
            _row_spec(BE, D),
            jax.ShapeDtypeStruct((epad, D), F32))(
                gr, gc, eap, w132, edge_b2[l].reshape(1, hdim),
                edge_W2[l], coord_W1[l], coord_b1[l].reshape(1, hdim),
                cw2p)

        s = sc_scatter(gout, rowp)

        h, coordp = _tc_call(
            _node_post_body, (gn,),
            [_row_spec(BN, D), _row_spec(BN, hdim), _row_spec(BN, 16),
             _row_spec(BN, 16), _full_spec(hdim, hdim), _full_spec(1, hdim),
             _full_spec(1, hdim), _full_spec(1, 1), _full_spec(hdim, hdim),
             _full_spec(hdim, hdim), _full_spec(1, hdim),
             _full_spec(hdim, hdim), _full_spec(1, hdim)],
            [_row_spec(BN, hdim), _row_spec(BN, 16)],
            [jax.ShapeDtypeStruct((n, hdim), F32),
             jax.ShapeDtypeStruct((n, 16), F32)])(
                s, h, coordp, velp,
                vel_W1[l], vel_b1[l].reshape(1, hdim),
                vel_W2[l].reshape(1, hdim), vel_b2[l].reshape(1, 1),
                node_W1[l][:hdim], node_W1[l][hdim:],
                node_b1[l].reshape(1, hdim), node_W2[l],
                node_b2[l].reshape(1, hdim))

    def hw(t):
        w1 = head_W1[t]
        h1 = w1[:hdim]
        c1 = jnp.pad(w1[hdim:hdim + 3], ((0, 13), (0, 0)))
        v1 = jnp.pad(w1[hdim + 3:hdim + 6], ((0, 13), (0, 0)))
        w3 = jnp.pad(head_W3[t], ((0, 0), (0, 5)))
        b3 = jnp.pad(head_b3[t], (0, 5)).reshape(1, 8)
        return (h1, c1, v1, head_b1[t].reshape(1, hdim), head_W2[t],
                head_b2[t].reshape(1, hdim), w3, b3)

    wspecs = [_full_spec(hdim, hdim), _full_spec(16, hdim),
              _full_spec(16, hdim), _full_spec(1, hdim),
              _full_spec(hdim, hdim), _full_spec(1, hdim),
              _full_spec(hdim, 8), _full_spec(1, 8)]
    out = _tc_call(
        _head_body, (gn,),
        [_row_spec(BN, hdim), _row_spec(BN, 16), _row_spec(BN, 16)]
        + wspecs + wspecs,
        _row_spec(BN, 16),
        jax.ShapeDtypeStruct((n, 16), F32))(
            h, coordp, velp, *hw(0), *hw(1))

    return out.reshape(n, nheads, 8)[:, :, :3].transpose(1, 0, 2)


# speedup vs baseline: 3.0254x; 1.1006x over previous
import functools

import jax
import jax.numpy as jnp
from jax import lax
from jax.experimental import pallas as pl
from jax.experimental.pallas import tpu as pltpu
from jax.experimental.pallas import tpu_sc as plsc

F32 = jnp.float32

TD = 128
D = 80
WIN = 1024
SWIN = 256
CHUNK = 128
NC, NS = 2, 16
NWORK = NC * NS

BN = 2000
BE = 2048
NCH = 5


def _silu(v):
    return v * jax.nn.sigmoid(v)


def _emb_body(x_ref, w_ref, b_ref, o_ref):
    o_ref[...] = jnp.dot(x_ref[...], w_ref[...],
                         preferred_element_type=F32) + b_ref[...]


def _node_pre_body(h_ref, cp_ref, w1a_ref, w1b_ref, b1_ref, tr_ref, tc_ref):
    h = h_ref[...]
    cp = cp_ref[...]
    z = jnp.zeros((h.shape[0], TD - 80), F32)
    u = jnp.dot(h, w1a_ref[...], preferred_element_type=F32) + b1_ref[...]
    v = jnp.dot(h, w1b_ref[...], preferred_element_type=F32)
    tr_ref[...] = jnp.concatenate([u, cp, z], axis=1)
    tc_ref[...] = jnp.concatenate([v, cp, z], axis=1)


def _edge_body(n_edges, eoff, gr_ref, gc_ref, ea_ref, w132_ref, b2_ref,
               w2_ref, cw1_ref, cb1_ref, cw2p_ref, o_ref):
    gr = gr_ref[...]
    gc = gc_ref[...]
    cd = gr[:, 64:80] - gc[:, 64:80]
    cat = jnp.concatenate([cd * cd, ea_ref[...]], axis=1)
    pre = (gr[:, :64] + gc[:, :64] +
           jnp.dot(cat, w132_ref[...], preferred_element_type=F32))
    m = _silu(jnp.dot(_silu(pre), w2_ref[...],
                      preferred_element_type=F32) + b2_ref[...])
    p = _silu(jnp.dot(m, cw1_ref[...], preferred_element_type=F32) + cb1_ref[...])
    cmat = jnp.dot(p, cw2p_ref[...], preferred_element_type=F32)[:, :1]
    trans = jnp.clip(cmat * cd, -100.0, 100.0)
    lane16 = lax.broadcasted_iota(jnp.int32, (1, 16), 1)
    tpack = trans + (lane16 == 3).astype(F32)
    base = eoff + pl.program_id(0) * gr.shape[0]
    valid = (lax.broadcasted_iota(jnp.int32, (gr.shape[0], 1), 0) + base
             < n_edges).astype(F32)
    o_ref[...] = jnp.concatenate([m, tpack], axis=1) * valid


def _node_post_body(s_ref, h_ref, cp_ref, vp_ref, vw1_ref, vb1_ref, vw2_ref,
                    vb2_ref, nw1h_ref, nw1a_ref, nb1_ref, nw2_ref, nb2_ref,
                    ho_ref, co_ref):
    s = s_ref[...]
    h = h_ref[...]
    cnt = jnp.clip(s[:, 67:68], 1.0, None)
    agg = s[:, :64] / cnt
    lane16 = lax.broadcasted_iota(jnp.int32, (1, 16), 1)
    dcoord = jnp.where(lane16 < 3, s[:, 64:80], 0.0) / cnt
    sv = _silu(jnp.dot(h, vw1_ref[...], preferred_element_type=F32) + vb1_ref[...])
    vmat = jnp.sum(sv * vw2_ref[...], axis=1, keepdims=True) + vb2_ref[...]
    co_ref[...] = cp_ref[...] + dcoord + vmat * vp_ref[...]
    z = _silu(jnp.dot(h, nw1h_ref[...], preferred_element_type=F32) +
              jnp.dot(agg, nw1a_ref[...], preferred_element_type=F32) +
              nb1_ref[...])
    ho_ref[...] = h + jnp.dot(z, nw2_ref[...],
                              preferred_element_type=F32) + nb2_ref[...]


def _head_body(h_ref, cp_ref, vp_ref,
               ah1_ref, ac1_ref, av1_ref, ab1_ref, aw2_ref, ab2_ref,
               aw3_ref, ab3_ref,
               bh1_ref, bc1_ref, bv1_ref, bb1_ref, bw2_ref, bb2_ref,
               bw3_ref, bb3_ref, o_ref):
    h = h_ref[...]
    cp = cp_ref[...]
    vp = vp_ref[...]

    def head(h1, c1, v1, b1, w2, b2, w3, b3):
        z = _silu(jnp.dot(h, h1, preferred_element_type=F32) +
                  jnp.dot(cp, c1, preferred_element_type=F32) +
                  jnp.dot(vp, v1, preferred_element_type=F32) + b1)
        z = _silu(jnp.dot(z, w2, preferred_element_type=F32) + b2)
        return jnp.dot(z, w3, preferred_element_type=F32) + b3

    oa = head(ah1_ref[...], ac1_ref[...], av1_ref[...], ab1_ref[...],
              aw2_ref[...], ab2_ref[...], aw3_ref[...], ab3_ref[...])
    ob = head(bh1_ref[...], bc1_ref[...], bv1_ref[...], bb1_ref[...],
              bw2_ref[...], bb2_ref[...], bw3_ref[...], bb3_ref[...])
    o_ref[...] = jnp.concatenate([oa, ob], axis=1)


def _tc_call(body, grid, in_specs, out_specs, out_shape):
    return pl.pallas_call(body, grid=grid, in_specs=in_specs,
                          out_specs=out_specs, out_shape=out_shape)


def _row_spec(b, d):
    return pl.BlockSpec((b, d), lambda i: (i, 0))


def _full_spec(s0, s1):
    return pl.BlockSpec((s0, s1), lambda i: (0, 0))


GR = 6
GDEPTH = 3


def _sc_gather_body(epad, tr_hbm, tc_hbm, row_hbm, col_hbm, gr_hbm, gc_hbm,
                    *refs):
    idxr_v, idxc_v = refs[0], refs[1]
    bufs = refs[2:2 + GR]
    gsems = refs[2 + GR:2 + 2 * GR]
    wsems = refs[2 + 2 * GR:2 + 3 * GR]
    core = lax.axis_index("c")
    sub = lax.axis_index("s")
    wid = sub * NC + core
    nwin = epad // WIN
    k = WIN // CHUNK

    @pl.loop(0, nwin // NWORK)
    def _(i):
        win = wid + i * NWORK
        pltpu.sync_copy(row_hbm.at[pl.ds(win * k, k)], idxr_v)
        pltpu.sync_copy(col_hbm.at[pl.ds(win * k, k)], idxc_v)

        sides = ((idxr_v, tr_hbm, gr_hbm), (idxc_v, tc_hbm, gc_hbm))
        pend_g = [None] * GR
        pend_w = [None] * GR

        def write_back(s):
            r = s % GR
            pend_g[r].wait()
            pend_g[r] = None
            out = sides[s // k][2]
            off = win * WIN + (s % k) * CHUNK
            pend_w[r] = pltpu.async_copy(bufs[r], out.at[pl.ds(off, CHUNK)],
                                         wsems[r])

        for s in range(2 * k):
            r = s % GR
            if pend_w[r] is not None:
                pend_w[r].wait()
                pend_w[r] = None
            idx_v, table, _ = sides[s // k]
            pend_g[r] = pltpu.async_copy(table.at[idx_v.at[s % k]], bufs[r],
                                         gsems[r])
            if s >= GDEPTH:
                write_back(s - GDEPTH)
        for s in range(2 * k - GDEPTH, 2 * k):
            write_back(s)
        for r in range(GR):
            if pend_w[r] is not None:
                pend_w[r].wait()


def _sc_scatter_body(n_half, q0, nch, *args):
    gouts = args[:nch]
    (row_hbm, s_hbm, idxa_v, idxb_v, bufa, bufb, acc,
     sua, sub_sem, ssa, ssb) = args[nch:]
    core = lax.axis_index("c")
    sub = lax.axis_index("s")
    echunk = gouts[0].shape[0]
    nwin = echunk // SWIN
    k = SWIN // CHUNK
    vz = jnp.zeros((16,), F32)
    iota = lax.iota(jnp.int32, 16)

    def zero_buf():
        @pl.loop(0, CHUNK)
        def _(r):
            @pl.loop(0, D // 16)
            def _(c):
                bufa[r, pl.ds(c * 16, 16)] = vz

    zero_buf()

    for p, (poff, psize) in enumerate(((0, q0), (q0, n_half - q0))):
        base = core * n_half + poff
        nchunk = pl.cdiv(psize + CHUNK, CHUNK)

        @pl.loop(0, pl.cdiv(nchunk, NS))
        def _(i):
            c = sub + i * NS

            @pl.when(c < nchunk)
            def _():
                pltpu.sync_copy(bufa.at[pl.ds(0, CHUNK)],
                                acc.at[pl.ds(c * CHUNK, CHUNK)])

        plsc.subcore_barrier()

        def remap(idx_v):
            @pl.loop(0, k)
            def _(j):
                @pl.loop(0, CHUNK // 16)
                def _(t):
                    v = idx_v[j, pl.ds(t * 16, 16)]
                    local = v - base
                    oob = (local < 0) | (local >= psize)
                    dump = psize + ((j * (CHUNK // 16) + t) % 5) * 16 + iota
                    idx_v[j, pl.ds(t * 16, 16)] = jnp.where(oob, dump, local)

        for c, gout_hbm in enumerate(gouts):
            irow0 = c * (echunk // CHUNK)

            @pl.loop(0, nwin // NS // 2)
            def _(i):
                w0 = sub + (2 * i) * NS
                w1 = sub + (2 * i + 1) * NS
                pltpu.sync_copy(row_hbm.at[pl.ds(irow0 + w0 * k, k)], idxa_v)
                ha = pltpu.async_copy(gout_hbm.at[pl.ds(w0 * SWIN, SWIN)],
                                      bufa, sua)
                pltpu.sync_copy(row_hbm.at[pl.ds(irow0 + w1 * k, k)], idxb_v)
                hb = pltpu.async_copy(gout_hbm.at[pl.ds(w1 * SWIN, SWIN)],
                                      bufb, sub_sem)
                remap(idxa_v)
                remap(idxb_v)
                ha.wait()
                adds_a = [pltpu.async_copy(bufa.at[pl.ds(j * CHUNK, CHUNK)],
                                           acc.at[idxa_v.at[j]], ssa, add=True)
                          for j in range(k)]
                hb.wait()
                adds_b = [pltpu.async_copy(bufb.at[pl.ds(j * CHUNK, CHUNK)],
                                           acc.at[idxb_v.at[j]], ssb, add=True)
                          for j in range(k)]
                for h in adds_a + adds_b:
                    h.wait()

        plsc.subcore_barrier()

        nfull = psize // CHUNK
        rem = psize - nfull * CHUNK

        @pl.loop(0, pl.cdiv(nfull, NS))
        def _(i):
            c = sub + i * NS

            @pl.when(c < nfull)
            def _():
                pltpu.sync_copy(acc.at[pl.ds(c * CHUNK, CHUNK)],
                                s_hbm.at[pl.ds(base + c * CHUNK, CHUNK)])

        if rem:
            @pl.when(sub == 0)
            def _():
                pltpu.sync_copy(acc.at[pl.ds(nfull * CHUNK, rem)],
                                s_hbm.at[pl.ds(base + nfull * CHUNK, rem)])

        if p == 0:
            plsc.subcore_barrier()
            zero_buf()


def kernel(x, pos, vel, edge_index, edge_attr, emb_W, emb_b,
           edge_W1, edge_b1, edge_W2, edge_b2,
           node_W1, node_b1, node_W2, node_b2,
           coord_W1, coord_b1, coord_W2,
           vel_W1, vel_b1, vel_W2, vel_b2,
           head_W1, head_b1, head_W2, head_b2, head_W3, head_b3):
    n, din = x.shape
    hdim = emb_W.shape[1]
    e = edge_index.shape[1]
    nlayers = edge_W1.shape[0]
    nheads = head_W1.shape[0]

    stride = WIN * NWORK
    epad = pl.cdiv(e, stride) * stride
    nch = NCH if (epad // stride) % NCH == 0 else 1
    echunk = epad // nch
    n_half = pl.cdiv(n, NC)
    q0 = pl.cdiv(n_half // 2, CHUNK) * CHUNK
    acc_rows = q0 + CHUNK

    rowp = jnp.pad(edge_index[0], (0, epad - e)).reshape(-1, CHUNK)
    colp = jnp.pad(edge_index[1], (0, epad - e)).reshape(-1, CHUNK)
    eap = jnp.pad(edge_attr, ((0, epad - e), (0, 0)))
    coordp = jnp.pad(pos, ((0, 0), (0, 16 - pos.shape[1])))
    velp = jnp.pad(vel, ((0, 0), (0, 16 - vel.shape[1])))

    gn = pl.cdiv(n, BN)
    ge = echunk // BE

    h = _tc_call(_emb_body, (gn,),
                 [_row_spec(BN, din), _full_spec(din, hdim),
                  _full_spec(1, hdim)],
                 _row_spec(BN, hdim),
                 jax.ShapeDtypeStruct((n, hdim), F32))(
                     x, emb_W, emb_b.reshape(1, hdim))

    mesh = plsc.VectorSubcoreMesh(core_axis_name="c", subcore_axis_name="s",
                                  num_cores=NC, num_subcores=NS)
    sc_gather = pl.kernel(
        functools.partial(_sc_gather_body, echunk),
        out_type=[jax.ShapeDtypeStruct((echunk, TD), F32),
                  jax.ShapeDtypeStruct((echunk, TD), F32)],
        mesh=mesh,
        scratch_types=([pltpu.VMEM((WIN // CHUNK, CHUNK), jnp.int32),
                        pltpu.VMEM((WIN // CHUNK, CHUNK), jnp.int32)]
                       + [pltpu.VMEM((CHUNK, TD), F32)] * GR
                       + [pltpu.SemaphoreType.DMA] * (2 * GR)))
    sc_scatter = pl.kernel(
        functools.partial(_sc_scatter_body, n_half, q0, nch),
        out_type=jax.ShapeDtypeStruct((n, D), F32),
        mesh=mesh,
        scratch_types=[pltpu.VMEM((SWIN // CHUNK, CHUNK), jnp.int32),
                       pltpu.VMEM((SWIN // CHUNK, CHUNK), jnp.int32),
                       pltpu.VMEM((SWIN, D), F32),
                       pltpu.VMEM((SWIN, D), F32),
                       pltpu.VMEM_SHARED((acc_rows, D), F32),
                       pltpu.SemaphoreType.DMA, pltpu.SemaphoreType.DMA,
                       pltpu.SemaphoreType.DMA, pltpu.SemaphoreType.DMA],
        compiler_params=pltpu.CompilerParams(use_tc_tiling_on_sc=False))

    for l in range(nlayers):
        w1 = edge_W1[l]
        w1a, w1b = w1[:hdim], w1[hdim:2 * hdim]
        wr = w1[2 * hdim:2 * hdim + 1]
        w1d = w1[2 * hdim + 1:]
        w132 = jnp.concatenate([jnp.broadcast_to(wr, (16, hdim)), w1d], axis=0)
        cw2p = jnp.pad(coord_W2[l], ((0, 0), (0, 7)))

        tr, tc = _tc_call(
            _node_pre_body, (gn,),
            [_row_spec(BN, hdim), _row_spec(BN, 16), _full_spec(hdim, hdim),
             _full_spec(hdim, hdim), _full_spec(1, hdim)],
            [_row_spec(BN, TD), _row_spec(BN, TD)],
            [jax.ShapeDtypeStruct((n, TD), F32),
             jax.ShapeDtypeStruct((n, TD), F32)])(
                 h, coordp, w1a, w1b, edge_b1[l].reshape(1, hdim))

        irows = echunk // CHUNK
        gouts = []
        for c in range(nch):
            gr, gc = sc_gather(tr, tc,
                               lax.slice_in_dim(rowp, c * irows,
                                                (c + 1) * irows),
                               lax.slice_in_dim(colp, c * irows,
                                                (c + 1) * irows))
            gouts.append(_tc_call(
                functools.partial(_edge_body, e, c * echunk), (ge,),
                [_row_spec(BE, TD), _row_spec(BE, TD),
                 _row_spec(BE, eap.shape[1]),
                 _full_spec(32, hdim), _full_spec(1, hdim),
                 _full_spec(hdim, hdim), _full_spec(hdim, hdim),
                 _full_spec(1, hdim), _full_spec(hdim, 8)],
                _row_spec(BE, D),
                jax.ShapeDtypeStruct((echunk, D), F32))(
                    gr, gc,
                    lax.slice_in_dim(eap, c * echunk, (c + 1) * echunk),
                    w132, edge_b2[l].reshape(1, hdim),
                    edge_W2[l], coord_W1[l], coord_b1[l].reshape(1, hdim),
                    cw2p))

        s = sc_scatter(*gouts, rowp)

        h, coordp = _tc_call(
            _node_post_body, (gn,),
            [_row_spec(BN, D), _row_spec(BN, hdim), _row_spec(BN, 16),
             _row_spec(BN, 16), _full_spec(hdim, hdim), _full_spec(1, hdim),
             _full_spec(1, hdim), _full_spec(1, 1), _full_spec(hdim, hdim),
             _full_spec(hdim, hdim), _full_spec(1, hdim),
             _full_spec(hdim, hdim), _full_spec(1, hdim)],
            [_row_spec(BN, hdim), _row_spec(BN, 16)],
            [jax.ShapeDtypeStruct((n, hdim), F32),
             jax.ShapeDtypeStruct((n, 16), F32)])(
                s, h, coordp, velp,
                vel_W1[l], vel_b1[l].reshape(1, hdim),
                vel_W2[l].reshape(1, hdim), vel_b2[l].reshape(1, 1),
                node_W1[l][:hdim], node_W1[l][hdim:],
                node_b1[l].reshape(1, hdim), node_W2[l],
                node_b2[l].reshape(1, hdim))

    def hw(t):
        w1 = head_W1[t]
        h1 = w1[:hdim]
        c1 = jnp.pad(w1[hdim:hdim + 3], ((0, 13), (0, 0)))
        v1 = jnp.pad(w1[hdim + 3:hdim + 6], ((0, 13), (0, 0)))
        w3 = jnp.pad(head_W3[t], ((0, 0), (0, 5)))
        b3 = jnp.pad(head_b3[t], (0, 5)).reshape(1, 8)
        return (h1, c1, v1, head_b1[t].reshape(1, hdim), head_W2[t],
                head_b2[t].reshape(1, hdim), w3, b3)

    wspecs = [_full_spec(hdim, hdim), _full_spec(16, hdim),
              _full_spec(16, hdim), _full_spec(1, hdim),
              _full_spec(hdim, hdim), _full_spec(1, hdim),
              _full_spec(hdim, 8), _full_spec(1, 8)]
    out = _tc_call(
        _head_body, (gn,),
        [_row_spec(BN, hdim), _row_spec(BN, 16), _row_spec(BN, 16)]
        + wspecs + wspecs,
        _row_spec(BN, 16),
        jax.ShapeDtypeStruct((n, 16), F32))(
            h, coordp, velp, *hw(0), *hw(1))

    return out.reshape(n, nheads, 8)[:, :, :3].transpose(1, 0, 2)
